# Initial kernel scaffold; baseline (speedup 1.0000x reference)
import jax
import jax.numpy as jnp
from jax.experimental import pallas as pl


def kernel(x, edge_index, Wl_0_0, bl_0_0, Wr_0_0, Wl_0_1, bl_0_1, Wr_0_1, Wl_0_2, bl_0_2, Wr_0_2, bn_g_0, bn_b_0, Wl_1_0, bl_1_0, Wr_1_0, Wl_1_1, bl_1_1, Wr_1_1, Wl_1_2, bl_1_2, Wr_1_2, bn_g_1, bn_b_1, clf_W, clf_b):
    raise NotImplementedError("write your pallas kernel here")


# speedup vs baseline: 3.0703x; 3.0703x over previous
import functools

import jax
import jax.numpy as jnp
from jax import lax
from jax.experimental import pallas as pl
from jax.experimental.pallas import tpu as pltpu
from jax.experimental.pallas import tpu_sc as plsc

N = 10000
E = 320000
D_IN = 128
HID = 128
NC, NS = 2, 16
NW = NC * NS
NB = 2
NV = NW * NB
R = 157
NPAD = NV * R
W = 128
EBLK = 256
KSC = 4000
FLUSH = 4096
BUF = 8704
CAP = E + 2 * FLUSH
ACC = (R + 1) * W
NEG = float("-inf")
POS = float("inf")

_mesh = plsc.VectorSubcoreMesh(core_axis_name="c", subcore_axis_name="s")
_sc_params = pltpu.CompilerParams(needs_layout_passes=False)


def _wid():
    return lax.axis_index("s") * NC + lax.axis_index("c")


def _bin_body(edges, srcl, ldstl, cntl,
              s_scan, d_scan, s_buf0, d_buf0, s_buf1, d_buf1, cbuf):
    wid = _wid()
    base = wid * (NB * R)
    iota = lax.iota(jnp.int32, 16)
    sbufs = (s_buf0, s_buf1)
    dbufs = (d_buf0, d_buf1)

    def chunk(k, carry):
        t0, w0, t1, w1 = carry
        pltpu.sync_copy(edges.at[pl.ds(k * KSC, KSC)], s_scan)
        pltpu.sync_copy(edges.at[pl.ds(E + k * KSC, KSC)], d_scan)

        def inner(i, wps):
            w0, w1 = wps
            s = s_scan[pl.ds(i * 16, 16)]
            d = d_scan[pl.ds(i * 16, 16)]
            dl = d - base
            m0 = (dl >= 0) & (dl < R)
            m1 = (dl >= R) & (dl < 2 * R)
            ps0 = plsc.cumsum(jnp.where(m0, 1, 0))
            ps1 = plsc.cumsum(jnp.where(m1, 1, 0))
            plsc.store_scatter(s_buf0, [w0 + ps0 - 1], s, mask=m0)
            plsc.store_scatter(d_buf0, [w0 + ps0 - 1], dl, mask=m0)
            plsc.store_scatter(s_buf1, [w1 + ps1 - 1], s, mask=m1)
            plsc.store_scatter(d_buf1, [w1 + ps1 - 1], dl - R, mask=m1)
            return w0 + ps0[15], w1 + ps1[15]

        w0, w1 = lax.fori_loop(0, KSC // 16, inner, (w0, w1))

        totals = [t0, t1]
        wps = [w0, w1]
        for b in range(NB):
            v = wid * NB + b
            do_flush = wps[b] >= FLUSH
            tot = totals[b]
            sb, db = sbufs[b], dbufs[b]

            @pl.when(do_flush)
            def _(tot=tot, sb=sb, db=db, v=v):
                toff = pl.multiple_of(tot, 8)
                pltpu.sync_copy(sb.at[pl.ds(0, FLUSH)],
                                srcl.at[pl.ds(v * CAP + toff, FLUSH)])
                pltpu.sync_copy(db.at[pl.ds(0, FLUSH)],
                                ldstl.at[pl.ds(v * CAP + toff, FLUSH)])

                def shift(i, _):
                    sb[pl.ds(i * 16, 16)] = sb[pl.ds(FLUSH + i * 16, 16)]
                    db[pl.ds(i * 16, 16)] = db[pl.ds(FLUSH + i * 16, 16)]
                    return 0

                lax.fori_loop(0, (BUF - FLUSH) // 16, shift, 0)

            totals[b] = jnp.where(do_flush, tot + FLUSH, tot)
            wps[b] = jnp.where(do_flush, wps[b] - FLUSH, wps[b])
        return totals[0], wps[0], totals[1], wps[1]

    z = jnp.int32(0)
    t0, w0, t1, w1 = lax.fori_loop(0, E // KSC, chunk, (z, z, z, z))

    ones = jnp.full((16,), True)
    totals = [t0, t1]
    wps = [w0, w1]
    for b in range(NB):
        v = wid * NB + b
        sent_s = v * R + iota
        sent_d = jnp.full((16,), R, jnp.int32)
        sb, db = sbufs[b], dbufs[b]
        wp = wps[b]
        for j in range(EBLK // 16):
            plsc.store_scatter(sb, [wp + j * 16 + iota], sent_s, mask=ones)
            plsc.store_scatter(db, [wp + j * 16 + iota], sent_d, mask=ones)
        wp = ((wp + EBLK - 1) // EBLK) * EBLK
        tot = totals[b]
        toff = pl.multiple_of(tot, 8)

        @pl.when(wp > 0)
        def _(sb=sb, db=db, toff=toff, v=v):
            pltpu.sync_copy(sb.at[pl.ds(0, FLUSH)],
                            srcl.at[pl.ds(v * CAP + toff, FLUSH)])
            pltpu.sync_copy(db.at[pl.ds(0, FLUSH)],
                            ldstl.at[pl.ds(v * CAP + toff, FLUSH)])

        @pl.when(wp > FLUSH)
        def _(sb=sb, db=db, toff=toff, v=v):
            pltpu.sync_copy(sb.at[pl.ds(FLUSH, FLUSH)],
                            srcl.at[pl.ds(v * CAP + toff + FLUSH, FLUSH)])
            pltpu.sync_copy(db.at[pl.ds(FLUSH, FLUSH)],
                            ldstl.at[pl.ds(v * CAP + toff + FLUSH, FLUSH)])

        cbuf[...] = jnp.full((16,), tot + wp, jnp.int32)
        pltpu.sync_copy(cbuf, cntl.at[pl.ds(v * 16, 16)])


_bin_edges = functools.partial(
    pl.kernel,
    out_type=[
        jax.ShapeDtypeStruct((NV * CAP,), jnp.int32),
        jax.ShapeDtypeStruct((NV * CAP,), jnp.int32),
        jax.ShapeDtypeStruct((NV * 16,), jnp.int32),
    ],
    mesh=_mesh,
    scratch_types=[
        pltpu.VMEM((KSC,), jnp.int32),
        pltpu.VMEM((KSC,), jnp.int32),
        pltpu.VMEM((BUF,), jnp.int32),
        pltpu.VMEM((BUF,), jnp.int32),
        pltpu.VMEM((BUF,), jnp.int32),
        pltpu.VMEM((BUF,), jnp.int32),
        pltpu.VMEM((16,), jnp.int32),
    ],
    compiler_params=_sc_params,
)(_bin_body)


def _agg_body(ncw, c, with_deg, tbl, srcl, ldstl, cntl, *refs):
    if with_deg:
        (osum, omax, omin, odeg, srcb, ldstb, idx2, gbuf,
         accs, accm, accn, degb, cntb, sem) = refs
    else:
        (osum, omax, omin, srcb, ldstb, idx2, gbuf,
         accs, accm, accn, cntb, sem) = refs
    wid = _wid()

    zero = jnp.zeros((16,), jnp.float32)
    one0 = jnp.where(lax.iota(jnp.int32, 16) == 0, 1.0, 0.0)

    for b in range(NB):
        v = wid * NB + b

        def initr(i, _):
            sl = pl.ds(i * 16, 16)
            accs[sl] = zero
            accm[sl] = zero + NEG
            accn[sl] = zero + POS
            return 0

        lax.fori_loop(0, ACC // 16, initr, 0)
        if with_deg:
            for i in range(176 // 16):
                degb[pl.ds(i * 16, 16)] = zero

        pltpu.sync_copy(cntl.at[pl.ds(v * 16, 16)], cntb)
        cnt = cntb[pl.ds(0, 16)][0]
        nblk = cnt // EBLK

        def blk(bi, _):
            boff = pl.multiple_of(bi * EBLK, 8)
            pltpu.sync_copy(srcl.at[pl.ds(v * CAP + boff, EBLK)], srcb)
            pltpu.sync_copy(ldstl.at[pl.ds(v * CAP + boff, EBLK)], ldstb)
            for i in range(EBLK // 16):
                vv = srcb[pl.ds(i * 16, 16)] * ncw + c
                idx2[i // 8, pl.ds((i % 8) * 16, 16)] = vv
            for j in range(EBLK // 128):
                pltpu.async_copy(tbl.at[idx2.at[j]],
                                 gbuf.at[pl.ds(j * 128, 128)], sem).wait()

            def gbody(g, _):
                dvec = ldstb[pl.ds(g * 16, 16)]
                for lane in range(16):
                    d = dvec[lane]
                    e = g * 16 + lane
                    for j in range(W // 16):
                        sl = pl.ds(d * W + j * 16, 16)
                        r = gbuf[e, pl.ds(j * 16, 16)]
                        accs[sl] = accs[sl] + r
                        accm[sl] = jnp.maximum(accm[sl], r)
                        accn[sl] = jnp.minimum(accn[sl], r)
                    if with_deg:
                        dsl = pl.ds(d, 16)
                        degb[dsl] = degb[dsl] + one0
                return 0

            lax.fori_loop(0, EBLK // 16, gbody, 0)
            return 0

        lax.fori_loop(0, nblk, blk, 0)

        pltpu.sync_copy(accs.at[pl.ds(0, R * W)],
                        osum.at[pl.ds(v * R * W, R * W)])
        pltpu.sync_copy(accm.at[pl.ds(0, R * W)],
                        omax.at[pl.ds(v * R * W, R * W)])
        pltpu.sync_copy(accn.at[pl.ds(0, R * W)],
                        omin.at[pl.ds(v * R * W, R * W)])
        if with_deg:
            pltpu.sync_copy(degb.at[pl.ds(0, 160)],
                            odeg.at[pl.ds(v * 160, 160)])


def _make_agg(ncw, c, with_deg):
    outs = [jax.ShapeDtypeStruct((NPAD * W,), jnp.float32)] * 3
    scratch = [
        pltpu.VMEM((EBLK,), jnp.int32),
        pltpu.VMEM((EBLK,), jnp.int32),
        pltpu.VMEM((EBLK // 128, 128), jnp.int32),
        pltpu.VMEM((EBLK, W), jnp.float32),
        pltpu.VMEM((ACC,), jnp.float32),
        pltpu.VMEM((ACC,), jnp.float32),
        pltpu.VMEM((ACC,), jnp.float32),
    ]
    if with_deg:
        outs = outs + [jax.ShapeDtypeStruct((NV * 160,), jnp.float32)]
        scratch = scratch + [pltpu.VMEM((176,), jnp.float32)]
    scratch = scratch + [pltpu.VMEM((16,), jnp.int32), pltpu.SemaphoreType.DMA]
    return pl.kernel(
        functools.partial(_agg_body, ncw, c, with_deg),
        out_type=outs,
        mesh=_mesh,
        scratch_types=scratch,
        compiler_params=_sc_params,
    )


BR = 2000


def _c1_body(h_ref, sm_ref, mx_ref, mn_ref, dg_ref,
             wl0, wr0, wl1, wr1, wl2, wr2, bl0, bl1, bl2,
             opre, ostat):
    i = pl.program_id(0)
    deg = dg_ref[...]
    degc = jnp.maximum(deg, 1.0)
    emp = deg <= 0.0
    h = h_ref[...]
    mean = sm_ref[...] / degc
    mxv = jnp.where(emp, 0.0, mx_ref[...])
    mnv = jnp.where(emp, 0.0, mn_ref[...])
    parts = []
    for agg, Wl, bl, Wr in ((mean, wl0, bl0, wr0),
                            (mxv, wl1, bl1, wr1),
                            (mnv, wl2, bl2, wr2)):
        parts.append(
            jnp.dot(agg, Wl[...], preferred_element_type=jnp.float32)
            + bl[...]
            + jnp.dot(h, Wr[...], preferred_element_type=jnp.float32))
    pre = jnp.concatenate(parts, axis=1)
    opre[...] = pre

    @pl.when(i == 0)
    def _():
        ostat[...] = jnp.zeros_like(ostat)

    s0 = jnp.sum(pre, axis=0)[None, :]
    s1 = jnp.sum(pre * pre, axis=0)[None, :]
    pad = jnp.zeros((6, pre.shape[1]), jnp.float32)
    ostat[...] = ostat[...] + jnp.concatenate([s0, s1, pad], axis=0)


def _make_c1(K):
    grid = N // BR
    rb = lambda i: (i, 0)
    cb = lambda i: (0, 0)
    return pl.pallas_call(
        _c1_body,
        grid=(grid,),
        in_specs=[
            pl.BlockSpec((BR, K), rb),
            pl.BlockSpec((BR, K), rb),
            pl.BlockSpec((BR, K), rb),
            pl.BlockSpec((BR, K), rb),
            pl.BlockSpec((BR, 1), rb),
        ] + [pl.BlockSpec((K, HID), cb)] * 6 + [pl.BlockSpec((1, HID), cb)] * 3,
        out_specs=[
            pl.BlockSpec((BR, 3 * HID), rb),
            pl.BlockSpec((8, 3 * HID), cb),
        ],
        out_shape=[
            jax.ShapeDtypeStruct((N, 3 * HID), jnp.float32),
            jax.ShapeDtypeStruct((8, 3 * HID), jnp.float32),
        ],
    )


def _c2_body(final, pre_ref, stat_ref, g_ref, b_ref, *rest):
    if final:
        cw_ref, cb_ref, out_ref = rest
    else:
        (out_ref,) = rest
    stat = stat_ref[...]
    mu = stat[0:1, :] / jnp.float32(N)
    var = stat[1:2, :] / jnp.float32(N) - mu * mu
    inv = lax.rsqrt(var + 1e-5)
    h = (pre_ref[...] - mu) * (inv * g_ref[...]) + b_ref[...]
    h = jnp.maximum(h, 0.0)
    if final:
        out_ref[...] = (jnp.dot(h, cw_ref[...],
                                preferred_element_type=jnp.float32)
                        + cb_ref[...])
    else:
        out_ref[...] = h


def _make_c2(final):
    grid = N // BR
    rb = lambda i: (i, 0)
    cb = lambda i: (0, 0)
    K = 3 * HID
    in_specs = [
        pl.BlockSpec((BR, K), rb),
        pl.BlockSpec((8, K), cb),
        pl.BlockSpec((1, K), cb),
        pl.BlockSpec((1, K), cb),
    ]
    if final:
        in_specs += [pl.BlockSpec((K, HID), cb), pl.BlockSpec((1, HID), cb)]
        out_w = HID
    else:
        out_w = K
    return pl.pallas_call(
        functools.partial(_c2_body, final),
        grid=(grid,),
        in_specs=in_specs,
        out_specs=pl.BlockSpec((BR, out_w), rb),
        out_shape=jax.ShapeDtypeStruct((N, out_w), jnp.float32),
    )


def _layer_aggregate(tbl2d, ncw, srcl, ldstl, cnt, first):
    sums, maxs, mins = [], [], []
    deg = None
    for c in range(ncw):
        agg = _make_agg(ncw, c, first and c == 0)
        if first and c == 0:
            s, m, n, deg = agg(tbl2d, srcl, ldstl, cnt)
        else:
            s, m, n = agg(tbl2d, srcl, ldstl, cnt)
        sums.append(s.reshape(NPAD, W))
        maxs.append(m.reshape(NPAD, W))
        mins.append(n.reshape(NPAD, W))
    sm = jnp.concatenate(sums, axis=1)[:N]
    mx = jnp.concatenate(maxs, axis=1)[:N]
    mn = jnp.concatenate(mins, axis=1)[:N]
    return sm, mx, mn, deg


def kernel(x, edge_index,
           Wl_0_0, bl_0_0, Wr_0_0,
           Wl_0_1, bl_0_1, Wr_0_1,
           Wl_0_2, bl_0_2, Wr_0_2,
           bn_g_0, bn_b_0,
           Wl_1_0, bl_1_0, Wr_1_0,
           Wl_1_1, bl_1_1, Wr_1_1,
           Wl_1_2, bl_1_2, Wr_1_2,
           bn_g_1, bn_b_1,
           clf_W, clf_b):
    srcl, ldstl, cnt = _bin_edges(edge_index.reshape(2 * E))

    sm0, mx0, mn0, deg = _layer_aggregate(x, D_IN // W, srcl, ldstl, cnt,
                                          True)
    degv = deg.reshape(NV, 160)[:, :R].reshape(NPAD, 1)[:N]
    c1 = _make_c1(D_IN)
    pre0, stat0 = c1(x, sm0, mx0, mn0, degv,
                     Wl_0_0, Wr_0_0, Wl_0_1, Wr_0_1, Wl_0_2, Wr_0_2,
                     bl_0_0.reshape(1, HID), bl_0_1.reshape(1, HID),
                     bl_0_2.reshape(1, HID))
    h1 = _make_c2(False)(pre0, stat0, bn_g_0.reshape(1, -1),
                         bn_b_0.reshape(1, -1))

    tbl1 = h1.reshape(N * (3 * HID // W), W)
    sm1, mx1, mn1, _ = _layer_aggregate(tbl1, 3 * HID // W, srcl, ldstl, cnt,
                                        False)
    c1b = _make_c1(3 * HID)
    pre1, stat1 = c1b(h1, sm1, mx1, mn1, degv,
                      Wl_1_0, Wr_1_0, Wl_1_1, Wr_1_1, Wl_1_2, Wr_1_2,
                      bl_1_0.reshape(1, HID), bl_1_1.reshape(1, HID),
                      bl_1_2.reshape(1, HID))
    clf_Wp = jnp.pad(clf_W, ((0, 0), (0, HID - clf_W.shape[1])))
    clf_bp = jnp.pad(clf_b, (0, HID - clf_b.shape[0])).reshape(1, HID)
    logits = _make_c2(True)(pre1, stat1, bn_g_1.reshape(1, -1),
                            bn_b_1.reshape(1, -1), clf_Wp, clf_bp)
    return logits[:, :clf_W.shape[1]]


# speedup vs baseline: 4.7496x; 1.5470x over previous
import functools

import jax
import jax.numpy as jnp
from jax import lax
from jax.experimental import pallas as pl
from jax.experimental.pallas import tpu as pltpu
from jax.experimental.pallas import tpu_sc as plsc

N = 10000
E = 320000
D_IN = 128
HID = 128
NC, NS = 2, 16
NW = NC * NS
NB = 2
NV = NW * NB
R = 157
NPAD = NV * R
W = 128
EBLK = 256
KSC = 4000
FLUSH = 4096
BUF = 8704
CAP = E + 2 * FLUSH
ACC = (R + 1) * W
NEG = float("-inf")
POS = float("inf")

_mesh = plsc.VectorSubcoreMesh(core_axis_name="c", subcore_axis_name="s")
_sc_params = pltpu.CompilerParams(needs_layout_passes=False)


def _wid():
    return lax.axis_index("s") * NC + lax.axis_index("c")


def _bin_body(edges, srcl, ldstl, cntl,
              s_scan, d_scan, s_buf0, d_buf0, s_buf1, d_buf1, cbuf):
    wid = _wid()
    base = wid * (NB * R)
    iota = lax.iota(jnp.int32, 16)
    sbufs = (s_buf0, s_buf1)
    dbufs = (d_buf0, d_buf1)

    def chunk(k, carry):
        t0, w0, t1, w1 = carry
        pltpu.sync_copy(edges.at[pl.ds(k * KSC, KSC)], s_scan)
        pltpu.sync_copy(edges.at[pl.ds(E + k * KSC, KSC)], d_scan)

        def inner(i, wps):
            w0, w1 = wps
            s = s_scan[pl.ds(i * 16, 16)]
            d = d_scan[pl.ds(i * 16, 16)]
            dl = d - base
            m0 = (dl >= 0) & (dl < R)
            m1 = (dl >= R) & (dl < 2 * R)
            ps0 = plsc.cumsum(jnp.where(m0, 1, 0))
            ps1 = plsc.cumsum(jnp.where(m1, 1, 0))
            plsc.store_scatter(s_buf0, [w0 + ps0 - 1], s, mask=m0)
            plsc.store_scatter(d_buf0, [w0 + ps0 - 1], dl, mask=m0)
            plsc.store_scatter(s_buf1, [w1 + ps1 - 1], s, mask=m1)
            plsc.store_scatter(d_buf1, [w1 + ps1 - 1], dl - R, mask=m1)
            return w0 + ps0[15], w1 + ps1[15]

        w0, w1 = lax.fori_loop(0, KSC // 16, inner, (w0, w1))

        totals = [t0, t1]
        wps = [w0, w1]
        for b in range(NB):
            v = wid * NB + b
            do_flush = wps[b] >= FLUSH
            tot = totals[b]
            sb, db = sbufs[b], dbufs[b]

            @pl.when(do_flush)
            def _(tot=tot, sb=sb, db=db, v=v):
                toff = pl.multiple_of(tot, 8)
                pltpu.sync_copy(sb.at[pl.ds(0, FLUSH)],
                                srcl.at[pl.ds(v * CAP + toff, FLUSH)])
                pltpu.sync_copy(db.at[pl.ds(0, FLUSH)],
                                ldstl.at[pl.ds(v * CAP + toff, FLUSH)])

                def shift(i, _):
                    sb[pl.ds(i * 16, 16)] = sb[pl.ds(FLUSH + i * 16, 16)]
                    db[pl.ds(i * 16, 16)] = db[pl.ds(FLUSH + i * 16, 16)]
                    return 0

                lax.fori_loop(0, (BUF - FLUSH) // 16, shift, 0)

            totals[b] = jnp.where(do_flush, tot + FLUSH, tot)
            wps[b] = jnp.where(do_flush, wps[b] - FLUSH, wps[b])
        return totals[0], wps[0], totals[1], wps[1]

    z = jnp.int32(0)
    t0, w0, t1, w1 = lax.fori_loop(0, E // KSC, chunk, (z, z, z, z))

    ones = jnp.full((16,), True)
    totals = [t0, t1]
    wps = [w0, w1]
    for b in range(NB):
        v = wid * NB + b
        sent_s = v * R + iota
        sent_d = jnp.full((16,), R, jnp.int32)
        sb, db = sbufs[b], dbufs[b]
        wp = wps[b]
        for j in range(EBLK // 16):
            plsc.store_scatter(sb, [wp + j * 16 + iota], sent_s, mask=ones)
            plsc.store_scatter(db, [wp + j * 16 + iota], sent_d, mask=ones)
        wp = ((wp + EBLK - 1) // EBLK) * EBLK
        tot = totals[b]
        toff = pl.multiple_of(tot, 8)

        @pl.when(wp > 0)
        def _(sb=sb, db=db, toff=toff, v=v):
            pltpu.sync_copy(sb.at[pl.ds(0, FLUSH)],
                            srcl.at[pl.ds(v * CAP + toff, FLUSH)])
            pltpu.sync_copy(db.at[pl.ds(0, FLUSH)],
                            ldstl.at[pl.ds(v * CAP + toff, FLUSH)])

        @pl.when(wp > FLUSH)
        def _(sb=sb, db=db, toff=toff, v=v):
            pltpu.sync_copy(sb.at[pl.ds(FLUSH, FLUSH)],
                            srcl.at[pl.ds(v * CAP + toff + FLUSH, FLUSH)])
            pltpu.sync_copy(db.at[pl.ds(FLUSH, FLUSH)],
                            ldstl.at[pl.ds(v * CAP + toff + FLUSH, FLUSH)])

        cbuf[...] = jnp.full((16,), tot + wp, jnp.int32)
        pltpu.sync_copy(cbuf, cntl.at[pl.ds(v * 16, 16)])


_bin_edges = functools.partial(
    pl.kernel,
    out_type=[
        jax.ShapeDtypeStruct((NV * CAP,), jnp.int32),
        jax.ShapeDtypeStruct((NV * CAP,), jnp.int32),
        jax.ShapeDtypeStruct((NV * 16,), jnp.int32),
    ],
    mesh=_mesh,
    scratch_types=[
        pltpu.VMEM((KSC,), jnp.int32),
        pltpu.VMEM((KSC,), jnp.int32),
        pltpu.VMEM((BUF,), jnp.int32),
        pltpu.VMEM((BUF,), jnp.int32),
        pltpu.VMEM((BUF,), jnp.int32),
        pltpu.VMEM((BUF,), jnp.int32),
        pltpu.VMEM((16,), jnp.int32),
    ],
    compiler_params=_sc_params,
)(_bin_body)


CAPS = 49152
SBLK = 4096


def _sort_body(srcl, ldstl, cntl, osrcl, oldstl, odeg,
               sblk, dblk, osrc, odst, histv, offv, degf, cntb):
    wid = _wid()
    zero_i = jnp.zeros((16,), jnp.int32)
    one0 = jnp.where(lax.iota(jnp.int32, 16) == 0, 1, 0)
    lane0 = lax.iota(jnp.int32, 16) == 0

    for b in range(NB):
        v = wid * NB + b
        for i in range(176 // 16):
            histv[pl.ds(i * 16, 16)] = zero_i

        pltpu.sync_copy(cntl.at[pl.ds(v * 16, 16)], cntb)
        cnt = cntb[pl.ds(0, 16)][0]
        nf = (cnt + SBLK - 1) // SBLK

        def hblk(bi, _):
            boff = pl.multiple_of(bi * SBLK, 8)
            pltpu.sync_copy(ldstl.at[pl.ds(v * CAP + boff, SBLK)], dblk)
            nin = jnp.minimum(cnt - bi * SBLK, SBLK)

            def hgrp(g, _):
                dvec = dblk[pl.ds(g * 16, 16)]
                for lane in range(16):
                    d = dvec[lane]
                    dsl = pl.ds(d, 16)
                    histv[dsl] = histv[dsl] + one0
                return 0

            lax.fori_loop(0, nin // 16, hgrp, 0)
            return 0

        lax.fori_loop(0, nf, hblk, 0)

        for i in range(160 // 16):
            degf[pl.ds(i * 16, 16)] = histv[pl.ds(i * 16, 16)].astype(
                jnp.float32)
        pltpu.sync_copy(degf, odeg.at[pl.ds(v * 160, 160)])

        carry = jnp.int32(0)
        for i in range(176 // 16):
            hv = histv[pl.ds(i * 16, 16)]
            ps = plsc.cumsum(hv)
            offv[pl.ds(i * 16, 16)] = ps - hv + carry
            carry = carry + ps[15]

        small = cnt <= CAPS

        @pl.when(small)
        def _(v=v, cnt=cnt, nf=nf):
            def pblk(bi, _):
                boff = pl.multiple_of(bi * SBLK, 8)
                pltpu.sync_copy(srcl.at[pl.ds(v * CAP + boff, SBLK)], sblk)
                pltpu.sync_copy(ldstl.at[pl.ds(v * CAP + boff, SBLK)], dblk)
                nin = jnp.minimum(cnt - bi * SBLK, SBLK)

                def pgrp(g, _):
                    dvec = dblk[pl.ds(g * 16, 16)]
                    svec = sblk[pl.ds(g * 16, 16)]
                    for lane in range(16):
                        d = dvec[lane]
                        s = svec[lane]
                        dsl = pl.ds(d, 16)
                        ov = offv[dsl]
                        p = ov[0]
                        offv[dsl] = ov + one0
                        pv = jnp.full((16,), p, jnp.int32)
                        plsc.store_scatter(osrc, [pv],
                                           jnp.full((16,), s, jnp.int32),
                                           mask=lane0)
                        plsc.store_scatter(odst, [pv],
                                           jnp.full((16,), d, jnp.int32),
                                           mask=lane0)
                    return 0

                lax.fori_loop(0, nin // 16, pgrp, 0)
                return 0

            lax.fori_loop(0, nf, pblk, 0)

            def wblk(bi, _):
                boff = pl.multiple_of(bi * SBLK, 8)
                pltpu.sync_copy(osrc.at[pl.ds(boff, SBLK)],
                                osrcl.at[pl.ds(v * CAP + boff, SBLK)])
                pltpu.sync_copy(odst.at[pl.ds(boff, SBLK)],
                                oldstl.at[pl.ds(v * CAP + boff, SBLK)])
                return 0

            lax.fori_loop(0, nf, wblk, 0)

        @pl.when(jnp.logical_not(small))
        def _(v=v, nf=nf):
            def cblk(bi, _):
                boff = pl.multiple_of(bi * SBLK, 8)
                pltpu.sync_copy(srcl.at[pl.ds(v * CAP + boff, SBLK)], sblk)
                pltpu.sync_copy(sblk,
                                osrcl.at[pl.ds(v * CAP + boff, SBLK)])
                pltpu.sync_copy(ldstl.at[pl.ds(v * CAP + boff, SBLK)], dblk)
                pltpu.sync_copy(dblk,
                                oldstl.at[pl.ds(v * CAP + boff, SBLK)])
                return 0

            lax.fori_loop(0, nf, cblk, 0)


_sort_bins = functools.partial(
    pl.kernel,
    out_type=[
        jax.ShapeDtypeStruct((NV * CAP,), jnp.int32),
        jax.ShapeDtypeStruct((NV * CAP,), jnp.int32),
        jax.ShapeDtypeStruct((NV * 160,), jnp.float32),
    ],
    mesh=_mesh,
    scratch_types=[
        pltpu.VMEM((SBLK,), jnp.int32),
        pltpu.VMEM((SBLK,), jnp.int32),
        pltpu.VMEM((CAPS,), jnp.int32),
        pltpu.VMEM((CAPS,), jnp.int32),
        pltpu.VMEM((176,), jnp.int32),
        pltpu.VMEM((176,), jnp.int32),
        pltpu.VMEM((160,), jnp.float32),
        pltpu.VMEM((16,), jnp.int32),
    ],
    compiler_params=_sc_params,
)(_sort_body)


def _agg_body(ncw, c, tbl, srcl, ldstl, cntl, *refs):
    (osum, omax, omin, srcb, ldstb, idx2, gbuf,
     accs, accm, accn, cntb, sem) = refs
    wid = _wid()

    zero = jnp.zeros((16,), jnp.float32)
    negs = zero + NEG
    poss = zero + POS
    NJ = W // 16

    for b in range(NB):
        v = wid * NB + b

        def initr(i, _):
            sl = pl.ds(i * 16, 16)
            accs[sl] = zero
            accm[sl] = negs
            accn[sl] = poss
            return 0

        lax.fori_loop(0, ACC // 16, initr, 0)

        pltpu.sync_copy(cntl.at[pl.ds(v * 16, 16)], cntb)
        cnt = cntb[pl.ds(0, 16)][0]
        nblk = cnt // EBLK

        def blk(bi, carry):
            boff = pl.multiple_of(bi * EBLK, 8)
            pltpu.sync_copy(srcl.at[pl.ds(v * CAP + boff, EBLK)], srcb)
            pltpu.sync_copy(ldstl.at[pl.ds(v * CAP + boff, EBLK)], ldstb)
            for i in range(EBLK // 16):
                vv = srcb[pl.ds(i * 16, 16)] * ncw + c
                idx2[i // 8, pl.ds((i % 8) * 16, 16)] = vv
            for j in range(EBLK // 128):
                pltpu.async_copy(tbl.at[idx2.at[j]],
                                 gbuf.at[pl.ds(j * 128, 128)], sem).wait()

            def gbody(g, carry2):
                dprev = carry2[0]
                regs = list(carry2[1:])
                dvec = ldstb[pl.ds(g * 16, 16)]
                for lane in range(16):
                    d = dvec[lane]
                    e = g * 16 + lane

                    def flush(args, dp=dprev):
                        for j in range(NJ):
                            sl = pl.ds(dp * W + j * 16, 16)
                            accs[sl] = accs[sl] + args[j]
                            accm[sl] = jnp.maximum(accm[sl], args[NJ + j])
                            accn[sl] = jnp.minimum(accn[sl], args[2 * NJ + j])
                        return ([zero] * NJ) + ([negs] * NJ) + ([poss] * NJ)

                    regs = lax.cond(d != dprev, flush, lambda a: list(a),
                                    tuple(regs))
                    regs = list(regs)
                    for j in range(NJ):
                        r = gbuf[e, pl.ds(j * 16, 16)]
                        regs[j] = regs[j] + r
                        regs[NJ + j] = jnp.maximum(regs[NJ + j], r)
                        regs[2 * NJ + j] = jnp.minimum(regs[2 * NJ + j], r)
                    dprev = d
                return (dprev, *regs)

            return lax.fori_loop(0, EBLK // 16, gbody, carry)

        init = (jnp.int32(R),) + tuple([zero] * NJ + [negs] * NJ + [poss] * NJ)
        fin = lax.fori_loop(0, nblk, blk, init)
        dlast = fin[0]
        for j in range(NJ):
            sl = pl.ds(dlast * W + j * 16, 16)
            accs[sl] = accs[sl] + fin[1 + j]
            accm[sl] = jnp.maximum(accm[sl], fin[1 + NJ + j])
            accn[sl] = jnp.minimum(accn[sl], fin[1 + 2 * NJ + j])

        pltpu.sync_copy(accs.at[pl.ds(0, R * W)],
                        osum.at[pl.ds(v * R * W, R * W)])
        pltpu.sync_copy(accm.at[pl.ds(0, R * W)],
                        omax.at[pl.ds(v * R * W, R * W)])
        pltpu.sync_copy(accn.at[pl.ds(0, R * W)],
                        omin.at[pl.ds(v * R * W, R * W)])


def _make_agg(ncw, c):
    outs = [jax.ShapeDtypeStruct((NPAD * W,), jnp.float32)] * 3
    scratch = [
        pltpu.VMEM((EBLK,), jnp.int32),
        pltpu.VMEM((EBLK,), jnp.int32),
        pltpu.VMEM((EBLK // 128, 128), jnp.int32),
        pltpu.VMEM((EBLK, W), jnp.float32),
        pltpu.VMEM((ACC,), jnp.float32),
        pltpu.VMEM((ACC,), jnp.float32),
        pltpu.VMEM((ACC,), jnp.float32),
        pltpu.VMEM((16,), jnp.int32),
        pltpu.SemaphoreType.DMA,
    ]
    return pl.kernel(
        functools.partial(_agg_body, ncw, c),
        out_type=outs,
        mesh=_mesh,
        scratch_types=scratch,
        compiler_params=_sc_params,
    )


BR = 2000


def _c1_body(h_ref, sm_ref, mx_ref, mn_ref, dg_ref,
             wl0, wr0, wl1, wr1, wl2, wr2, bl0, bl1, bl2,
             opre, ostat):
    i = pl.program_id(0)
    deg = dg_ref[...]
    degc = jnp.maximum(deg, 1.0)
    emp = deg <= 0.0
    h = h_ref[...]
    mean = sm_ref[...] / degc
    mxv = jnp.where(emp, 0.0, mx_ref[...])
    mnv = jnp.where(emp, 0.0, mn_ref[...])
    parts = []
    for agg, Wl, bl, Wr in ((mean, wl0, bl0, wr0),
                            (mxv, wl1, bl1, wr1),
                            (mnv, wl2, bl2, wr2)):
        parts.append(
            jnp.dot(agg, Wl[...], preferred_element_type=jnp.float32)
            + bl[...]
            + jnp.dot(h, Wr[...], preferred_element_type=jnp.float32))
    pre = jnp.concatenate(parts, axis=1)
    opre[...] = pre

    @pl.when(i == 0)
    def _():
        ostat[...] = jnp.zeros_like(ostat)

    s0 = jnp.sum(pre, axis=0)[None, :]
    s1 = jnp.sum(pre * pre, axis=0)[None, :]
    pad = jnp.zeros((6, pre.shape[1]), jnp.float32)
    ostat[...] = ostat[...] + jnp.concatenate([s0, s1, pad], axis=0)


def _make_c1(K):
    grid = N // BR
    rb = lambda i: (i, 0)
    cb = lambda i: (0, 0)
    return pl.pallas_call(
        _c1_body,
        grid=(grid,),
        in_specs=[
            pl.BlockSpec((BR, K), rb),
            pl.BlockSpec((BR, K), rb),
            pl.BlockSpec((BR, K), rb),
            pl.BlockSpec((BR, K), rb),
            pl.BlockSpec((BR, 1), rb),
        ] + [pl.BlockSpec((K, HID), cb)] * 6 + [pl.BlockSpec((1, HID), cb)] * 3,
        out_specs=[
            pl.BlockSpec((BR, 3 * HID), rb),
            pl.BlockSpec((8, 3 * HID), cb),
        ],
        out_shape=[
            jax.ShapeDtypeStruct((N, 3 * HID), jnp.float32),
            jax.ShapeDtypeStruct((8, 3 * HID), jnp.float32),
        ],
    )


def _c2_body(final, pre_ref, stat_ref, g_ref, b_ref, *rest):
    if final:
        cw_ref, cb_ref, out_ref = rest
    else:
        (out_ref,) = rest
    stat = stat_ref[...]
    mu = stat[0:1, :] / jnp.float32(N)
    var = stat[1:2, :] / jnp.float32(N) - mu * mu
    inv = lax.rsqrt(var + 1e-5)
    h = (pre_ref[...] - mu) * (inv * g_ref[...]) + b_ref[...]
    h = jnp.maximum(h, 0.0)
    if final:
        out_ref[...] = (jnp.dot(h, cw_ref[...],
                                preferred_element_type=jnp.float32)
                        + cb_ref[...])
    else:
        out_ref[...] = h


def _make_c2(final):
    grid = N // BR
    rb = lambda i: (i, 0)
    cb = lambda i: (0, 0)
    K = 3 * HID
    in_specs = [
        pl.BlockSpec((BR, K), rb),
        pl.BlockSpec((8, K), cb),
        pl.BlockSpec((1, K), cb),
        pl.BlockSpec((1, K), cb),
    ]
    if final:
        in_specs += [pl.BlockSpec((K, HID), cb), pl.BlockSpec((1, HID), cb)]
        out_w = HID
    else:
        out_w = K
    return pl.pallas_call(
        functools.partial(_c2_body, final),
        grid=(grid,),
        in_specs=in_specs,
        out_specs=pl.BlockSpec((BR, out_w), rb),
        out_shape=jax.ShapeDtypeStruct((N, out_w), jnp.float32),
    )


def _layer_aggregate(tbl2d, ncw, srcl, ldstl, cnt):
    sums, maxs, mins = [], [], []
    for c in range(ncw):
        s, m, n = _make_agg(ncw, c)(tbl2d, srcl, ldstl, cnt)
        sums.append(s.reshape(NPAD, W))
        maxs.append(m.reshape(NPAD, W))
        mins.append(n.reshape(NPAD, W))
    sm = jnp.concatenate(sums, axis=1)[:N]
    mx = jnp.concatenate(maxs, axis=1)[:N]
    mn = jnp.concatenate(mins, axis=1)[:N]
    return sm, mx, mn


def kernel(x, edge_index,
           Wl_0_0, bl_0_0, Wr_0_0,
           Wl_0_1, bl_0_1, Wr_0_1,
           Wl_0_2, bl_0_2, Wr_0_2,
           bn_g_0, bn_b_0,
           Wl_1_0, bl_1_0, Wr_1_0,
           Wl_1_1, bl_1_1, Wr_1_1,
           Wl_1_2, bl_1_2, Wr_1_2,
           bn_g_1, bn_b_1,
           clf_W, clf_b):
    srcl, ldstl, cnt = _bin_edges(edge_index.reshape(2 * E))
    srcl, ldstl, deg = _sort_bins(srcl, ldstl, cnt)

    sm0, mx0, mn0 = _layer_aggregate(x, D_IN // W, srcl, ldstl, cnt)
    degv = deg.reshape(NV, 160)[:, :R].reshape(NPAD, 1)[:N]
    c1 = _make_c1(D_IN)
    pre0, stat0 = c1(x, sm0, mx0, mn0, degv,
                     Wl_0_0, Wr_0_0, Wl_0_1, Wr_0_1, Wl_0_2, Wr_0_2,
                     bl_0_0.reshape(1, HID), bl_0_1.reshape(1, HID),
                     bl_0_2.reshape(1, HID))
    h1 = _make_c2(False)(pre0, stat0, bn_g_0.reshape(1, -1),
                         bn_b_0.reshape(1, -1))

    tbl1 = h1.reshape(N * (3 * HID // W), W)
    sm1, mx1, mn1 = _layer_aggregate(tbl1, 3 * HID // W, srcl, ldstl, cnt)
    c1b = _make_c1(3 * HID)
    pre1, stat1 = c1b(h1, sm1, mx1, mn1, degv,
                      Wl_1_0, Wr_1_0, Wl_1_1, Wr_1_1, Wl_1_2, Wr_1_2,
                      bl_1_0.reshape(1, HID), bl_1_1.reshape(1, HID),
                      bl_1_2.reshape(1, HID))
    clf_Wp = jnp.pad(clf_W, ((0, 0), (0, HID - clf_W.shape[1])))
    clf_bp = jnp.pad(clf_b, (0, HID - clf_b.shape[0])).reshape(1, HID)
    logits = _make_c2(True)(pre1, stat1, bn_g_1.reshape(1, -1),
                            bn_b_1.reshape(1, -1), clf_Wp, clf_bp)
    return logits[:, :clf_W.shape[1]]


# speedup vs baseline: 5.5988x; 1.1788x over previous
import functools

import jax
import jax.numpy as jnp
from jax import lax
from jax.experimental import pallas as pl
from jax.experimental.pallas import tpu as pltpu
from jax.experimental.pallas import tpu_sc as plsc

N = 10000
E = 320000
D_IN = 128
HID = 128
NC, NS = 2, 16
NW = NC * NS
NB = 2
NV = NW * NB
R = 157
NPAD = NV * R
W = 128
EBLK = 256
KSC = 4000
FLUSH = 4096
BUF = 8704
CAP = E + 2 * FLUSH
ACC = (R + 1) * W
NEG = float("-inf")
POS = float("inf")

_mesh = plsc.VectorSubcoreMesh(core_axis_name="c", subcore_axis_name="s")
_sc_params = pltpu.CompilerParams(needs_layout_passes=False)


def _wid():
    return lax.axis_index("s") * NC + lax.axis_index("c")


def _bin_body(edges, srcl, ldstl, cntl,
              s_scan, d_scan, s_buf0, d_buf0, s_buf1, d_buf1, cbuf):
    wid = _wid()
    base = wid * (NB * R)
    iota = lax.iota(jnp.int32, 16)
    sbufs = (s_buf0, s_buf1)
    dbufs = (d_buf0, d_buf1)

    def chunk(k, carry):
        t0, w0, t1, w1 = carry
        pltpu.sync_copy(edges.at[pl.ds(k * KSC, KSC)], s_scan)
        pltpu.sync_copy(edges.at[pl.ds(E + k * KSC, KSC)], d_scan)

        def inner(i, wps):
            w0, w1 = wps
            s = s_scan[pl.ds(i * 16, 16)]
            d = d_scan[pl.ds(i * 16, 16)]
            dl = d - base
            m0 = (dl >= 0) & (dl < R)
            m1 = (dl >= R) & (dl < 2 * R)
            ps0 = plsc.cumsum(jnp.where(m0, 1, 0))
            ps1 = plsc.cumsum(jnp.where(m1, 1, 0))
            plsc.store_scatter(s_buf0, [w0 + ps0 - 1], s, mask=m0)
            plsc.store_scatter(d_buf0, [w0 + ps0 - 1], dl, mask=m0)
            plsc.store_scatter(s_buf1, [w1 + ps1 - 1], s, mask=m1)
            plsc.store_scatter(d_buf1, [w1 + ps1 - 1], dl - R, mask=m1)
            return w0 + ps0[15], w1 + ps1[15]

        w0, w1 = lax.fori_loop(0, KSC // 16, inner, (w0, w1))

        totals = [t0, t1]
        wps = [w0, w1]
        for b in range(NB):
            v = wid * NB + b
            do_flush = wps[b] >= FLUSH
            tot = totals[b]
            sb, db = sbufs[b], dbufs[b]

            @pl.when(do_flush)
            def _(tot=tot, sb=sb, db=db, v=v):
                toff = pl.multiple_of(tot, 8)
                pltpu.sync_copy(sb.at[pl.ds(0, FLUSH)],
                                srcl.at[pl.ds(v * CAP + toff, FLUSH)])
                pltpu.sync_copy(db.at[pl.ds(0, FLUSH)],
                                ldstl.at[pl.ds(v * CAP + toff, FLUSH)])

                def shift(i, _):
                    sb[pl.ds(i * 16, 16)] = sb[pl.ds(FLUSH + i * 16, 16)]
                    db[pl.ds(i * 16, 16)] = db[pl.ds(FLUSH + i * 16, 16)]
                    return 0

                lax.fori_loop(0, (BUF - FLUSH) // 16, shift, 0)

            totals[b] = jnp.where(do_flush, tot + FLUSH, tot)
            wps[b] = jnp.where(do_flush, wps[b] - FLUSH, wps[b])
        return totals[0], wps[0], totals[1], wps[1]

    z = jnp.int32(0)
    t0, w0, t1, w1 = lax.fori_loop(0, E // KSC, chunk, (z, z, z, z))

    ones = jnp.full((16,), True)
    totals = [t0, t1]
    wps = [w0, w1]
    for b in range(NB):
        v = wid * NB + b
        sent_s = v * R + iota
        sent_d = jnp.full((16,), R, jnp.int32)
        sb, db = sbufs[b], dbufs[b]
        wp = wps[b]
        for j in range(EBLK // 16):
            plsc.store_scatter(sb, [wp + j * 16 + iota], sent_s, mask=ones)
            plsc.store_scatter(db, [wp + j * 16 + iota], sent_d, mask=ones)
        wp = ((wp + EBLK - 1) // EBLK) * EBLK
        tot = totals[b]
        toff = pl.multiple_of(tot, 8)

        @pl.when(wp > 0)
        def _(sb=sb, db=db, toff=toff, v=v):
            pltpu.sync_copy(sb.at[pl.ds(0, FLUSH)],
                            srcl.at[pl.ds(v * CAP + toff, FLUSH)])
            pltpu.sync_copy(db.at[pl.ds(0, FLUSH)],
                            ldstl.at[pl.ds(v * CAP + toff, FLUSH)])

        @pl.when(wp > FLUSH)
        def _(sb=sb, db=db, toff=toff, v=v):
            pltpu.sync_copy(sb.at[pl.ds(FLUSH, FLUSH)],
                            srcl.at[pl.ds(v * CAP + toff + FLUSH, FLUSH)])
            pltpu.sync_copy(db.at[pl.ds(FLUSH, FLUSH)],
                            ldstl.at[pl.ds(v * CAP + toff + FLUSH, FLUSH)])

        cbuf[...] = jnp.full((16,), tot + wp, jnp.int32)
        pltpu.sync_copy(cbuf, cntl.at[pl.ds(v * 16, 16)])


_bin_edges = functools.partial(
    pl.kernel,
    out_type=[
        jax.ShapeDtypeStruct((NV * CAP,), jnp.int32),
        jax.ShapeDtypeStruct((NV * CAP,), jnp.int32),
        jax.ShapeDtypeStruct((NV * 16,), jnp.int32),
    ],
    mesh=_mesh,
    scratch_types=[
        pltpu.VMEM((KSC,), jnp.int32),
        pltpu.VMEM((KSC,), jnp.int32),
        pltpu.VMEM((BUF,), jnp.int32),
        pltpu.VMEM((BUF,), jnp.int32),
        pltpu.VMEM((BUF,), jnp.int32),
        pltpu.VMEM((BUF,), jnp.int32),
        pltpu.VMEM((16,), jnp.int32),
    ],
    compiler_params=_sc_params,
)(_bin_body)


CAPS = 49152
SBLK = 4096


def _sort_body(srcl, ldstl, cntl, osrcl, ooffl, odeg,
               sblk, dblk, osrc, wo, histv, offv, degf, cntb):
    wid = _wid()
    zero_i = jnp.zeros((16,), jnp.int32)
    one0 = jnp.where(lax.iota(jnp.int32, 16) == 0, 1, 0)
    lane0 = lax.iota(jnp.int32, 16) == 0

    for b in range(NB):
        v = wid * NB + b
        for i in range(176 // 16):
            histv[pl.ds(i * 16, 16)] = zero_i

        pltpu.sync_copy(cntl.at[pl.ds(v * 16, 16)], cntb)
        cnt = cntb[pl.ds(0, 16)][0]
        nf = (cnt + SBLK - 1) // SBLK

        def hblk(bi, _):
            boff = pl.multiple_of(bi * SBLK, 8)
            pltpu.sync_copy(ldstl.at[pl.ds(v * CAP + boff, SBLK)], dblk)
            nin = jnp.minimum(cnt - bi * SBLK, SBLK)

            def hgrp(g, _):
                dvec = dblk[pl.ds(g * 16, 16)]
                for lane in range(16):
                    d = dvec[lane]
                    dsl = pl.ds(d, 16)
                    histv[dsl] = histv[dsl] + one0
                return 0

            lax.fori_loop(0, nin // 16, hgrp, 0)
            return 0

        lax.fori_loop(0, nf, hblk, 0)

        for i in range(160 // 16):
            degf[pl.ds(i * 16, 16)] = histv[pl.ds(i * 16, 16)].astype(
                jnp.float32)
        pltpu.sync_copy(degf, odeg.at[pl.ds(v * 160, 160)])

        carry = jnp.int32(0)
        for i in range(176 // 16):
            hv = histv[pl.ds(i * 16, 16)]
            ps = plsc.cumsum(hv)
            offv[pl.ds(i * 16, 16)] = ps - hv + carry
            carry = carry + ps[15]

        pltpu.sync_copy(offv.at[pl.ds(0, 160)], ooffl.at[pl.ds(v * 160, 160)])

        npass = (cnt + CAPS - 1) // CAPS

        def ppass(w, _):
            w0 = pl.multiple_of(w * CAPS, 8)
            for i in range(176 // 16):
                wo[pl.ds(i * 16, 16)] = offv[pl.ds(i * 16, 16)]

            def pblk(bi, _):
                boff = pl.multiple_of(bi * SBLK, 8)
                pltpu.sync_copy(srcl.at[pl.ds(v * CAP + boff, SBLK)], sblk)
                pltpu.sync_copy(ldstl.at[pl.ds(v * CAP + boff, SBLK)], dblk)
                nin = jnp.minimum(cnt - bi * SBLK, SBLK)

                def pgrp(g, _):
                    dvec = dblk[pl.ds(g * 16, 16)]
                    svec = sblk[pl.ds(g * 16, 16)]
                    for lane in range(16):
                        d = dvec[lane]
                        s = svec[lane]
                        dsl = pl.ds(d, 16)
                        ov = wo[dsl]
                        p = ov[0]
                        wo[dsl] = ov + one0
                        pw = p - w0
                        inwin = (pw >= 0) & (pw < CAPS)
                        mk = lane0 & jnp.full((16,), inwin)
                        pv = jnp.full((16,), pw, jnp.int32)
                        plsc.store_scatter(osrc, [pv],
                                           jnp.full((16,), s, jnp.int32),
                                           mask=mk)
                    return 0

                lax.fori_loop(0, nin // 16, pgrp, 0)
                return 0

            lax.fori_loop(0, nf, pblk, 0)

            nw = (jnp.minimum(cnt - w0, CAPS) + SBLK - 1) // SBLK

            def wblk(bi, _):
                boff = pl.multiple_of(bi * SBLK, 8)
                pltpu.sync_copy(osrc.at[pl.ds(boff, SBLK)],
                                osrcl.at[pl.ds(v * CAP + w0 + boff, SBLK)])
                return 0

            lax.fori_loop(0, nw, wblk, 0)
            return 0

        lax.fori_loop(0, npass, ppass, 0)


_sort_bins = functools.partial(
    pl.kernel,
    out_type=[
        jax.ShapeDtypeStruct((NV * CAP,), jnp.int32),
        jax.ShapeDtypeStruct((NV * 160,), jnp.int32),
        jax.ShapeDtypeStruct((NV * 160,), jnp.float32),
    ],
    mesh=_mesh,
    scratch_types=[
        pltpu.VMEM((SBLK,), jnp.int32),
        pltpu.VMEM((SBLK,), jnp.int32),
        pltpu.VMEM((CAPS,), jnp.int32),
        pltpu.VMEM((176,), jnp.int32),
        pltpu.VMEM((176,), jnp.int32),
        pltpu.VMEM((176,), jnp.int32),
        pltpu.VMEM((160,), jnp.float32),
        pltpu.VMEM((16,), jnp.int32),
    ],
    compiler_params=_sc_params,
)(_sort_body)


def _agg_body(ncw, c, tbl, srcl, ooffl, cntl, *refs):
    (osum, omax, omin, srcb, idx2, gbuf,
     accs, accm, accn, offb, cntb, sem) = refs
    wid = _wid()

    zero = jnp.zeros((16,), jnp.float32)
    negs = zero + NEG
    poss = zero + POS
    NJ = W // 16

    for b in range(NB):
        v = wid * NB + b

        def initr(i, _):
            sl = pl.ds(i * 16, 16)
            accs[sl] = zero
            accm[sl] = negs
            accn[sl] = poss
            return 0

        lax.fori_loop(0, ACC // 16, initr, 0)

        pltpu.sync_copy(cntl.at[pl.ds(v * 16, 16)], cntb)
        pltpu.sync_copy(ooffl.at[pl.ds(v * 160, 160)], offb.at[pl.ds(0, 160)])
        cnt = cntb[pl.ds(0, 16)][0]
        nblk = cnt // EBLK

        def blk(bi, carry):
            boff = pl.multiple_of(bi * EBLK, 8)
            pltpu.sync_copy(srcl.at[pl.ds(v * CAP + boff, EBLK)], srcb)
            for i in range(EBLK // 16):
                vv = srcb[pl.ds(i * 16, 16)] * ncw + c
                idx2[i // 8, pl.ds((i % 8) * 16, 16)] = vv
            for j in range(EBLK // 128):
                pltpu.async_copy(tbl.at[idx2.at[j]],
                                 gbuf.at[pl.ds(j * 128, 128)], sem).wait()
            e0 = bi * EBLK

            def seg_cond(st):
                return st[1] < EBLK

            def seg_body(st):
                r = st[0]
                pos = st[1]
                regs = list(st[2:])
                rend = offb[pl.ds(r + 1, 16)][0] - e0
                send = jnp.minimum(rend, EBLK)

                def acc_e(el, regs2):
                    regs2 = list(regs2)
                    for j in range(NJ):
                        rr = gbuf[el, pl.ds(j * 16, 16)]
                        regs2[j] = regs2[j] + rr
                        regs2[NJ + j] = jnp.maximum(regs2[NJ + j], rr)
                        regs2[2 * NJ + j] = jnp.minimum(regs2[2 * NJ + j], rr)
                    return tuple(regs2)

                regs = list(lax.fori_loop(pos, send, acc_e, tuple(regs)))
                fin = rend <= EBLK

                def flush(args, rr=r):
                    for j in range(NJ):
                        sl = pl.ds(rr * W + j * 16, 16)
                        accs[sl] = accs[sl] + args[j]
                        accm[sl] = jnp.maximum(accm[sl], args[NJ + j])
                        accn[sl] = jnp.minimum(accn[sl], args[2 * NJ + j])
                    return ([zero] * NJ) + ([negs] * NJ) + ([poss] * NJ)

                regs = lax.cond(fin, flush, lambda a: list(a), tuple(regs))
                r = jnp.where(fin, r + 1, r)
                return (r, send, *regs)

            st = lax.while_loop(seg_cond, seg_body,
                                (carry[0], jnp.int32(0), *carry[1:]))
            return (st[0], *st[2:])

        init = (jnp.int32(0),) + tuple([zero] * NJ + [negs] * NJ + [poss] * NJ)
        lax.fori_loop(0, nblk, blk, init)

        pltpu.sync_copy(accs.at[pl.ds(0, R * W)],
                        osum.at[pl.ds(v * R * W, R * W)])
        pltpu.sync_copy(accm.at[pl.ds(0, R * W)],
                        omax.at[pl.ds(v * R * W, R * W)])
        pltpu.sync_copy(accn.at[pl.ds(0, R * W)],
                        omin.at[pl.ds(v * R * W, R * W)])


def _make_agg(ncw, c):
    outs = [jax.ShapeDtypeStruct((NPAD * W,), jnp.float32)] * 3
    scratch = [
        pltpu.VMEM((EBLK,), jnp.int32),
        pltpu.VMEM((EBLK // 128, 128), jnp.int32),
        pltpu.VMEM((EBLK, W), jnp.float32),
        pltpu.VMEM((ACC,), jnp.float32),
        pltpu.VMEM((ACC,), jnp.float32),
        pltpu.VMEM((ACC,), jnp.float32),
        pltpu.VMEM((176,), jnp.int32),
        pltpu.VMEM((16,), jnp.int32),
        pltpu.SemaphoreType.DMA,
    ]
    return pl.kernel(
        functools.partial(_agg_body, ncw, c),
        out_type=outs,
        mesh=_mesh,
        scratch_types=scratch,
        compiler_params=_sc_params,
    )


BR = 2000


def _c1_body(h_ref, sm_ref, mx_ref, mn_ref, dg_ref,
             wl0, wr0, wl1, wr1, wl2, wr2, bl0, bl1, bl2,
             opre, ostat):
    i = pl.program_id(0)
    deg = dg_ref[...]
    degc = jnp.maximum(deg, 1.0)
    emp = deg <= 0.0
    h = h_ref[...]
    mean = sm_ref[...] / degc
    mxv = jnp.where(emp, 0.0, mx_ref[...])
    mnv = jnp.where(emp, 0.0, mn_ref[...])
    parts = []
    for agg, Wl, bl, Wr in ((mean, wl0, bl0, wr0),
                            (mxv, wl1, bl1, wr1),
                            (mnv, wl2, bl2, wr2)):
        parts.append(
            jnp.dot(agg, Wl[...], preferred_element_type=jnp.float32)
            + bl[...]
            + jnp.dot(h, Wr[...], preferred_element_type=jnp.float32))
    pre = jnp.concatenate(parts, axis=1)
    opre[...] = pre

    @pl.when(i == 0)
    def _():
        ostat[...] = jnp.zeros_like(ostat)

    s0 = jnp.sum(pre, axis=0)[None, :]
    s1 = jnp.sum(pre * pre, axis=0)[None, :]
    pad = jnp.zeros((6, pre.shape[1]), jnp.float32)
    ostat[...] = ostat[...] + jnp.concatenate([s0, s1, pad], axis=0)


def _make_c1(K):
    grid = N // BR
    rb = lambda i: (i, 0)
    cb = lambda i: (0, 0)
    return pl.pallas_call(
        _c1_body,
        grid=(grid,),
        in_specs=[
            pl.BlockSpec((BR, K), rb),
            pl.BlockSpec((BR, K), rb),
            pl.BlockSpec((BR, K), rb),
            pl.BlockSpec((BR, K), rb),
            pl.BlockSpec((BR, 1), rb),
        ] + [pl.BlockSpec((K, HID), cb)] * 6 + [pl.BlockSpec((1, HID), cb)] * 3,
        out_specs=[
            pl.BlockSpec((BR, 3 * HID), rb),
            pl.BlockSpec((8, 3 * HID), cb),
        ],
        out_shape=[
            jax.ShapeDtypeStruct((N, 3 * HID), jnp.float32),
            jax.ShapeDtypeStruct((8, 3 * HID), jnp.float32),
        ],
    )


def _c2_body(final, pre_ref, stat_ref, g_ref, b_ref, *rest):
    if final:
        cw_ref, cb_ref, out_ref = rest
    else:
        (out_ref,) = rest
    stat = stat_ref[...]
    mu = stat[0:1, :] / jnp.float32(N)
    var = stat[1:2, :] / jnp.float32(N) - mu * mu
    inv = lax.rsqrt(var + 1e-5)
    h = (pre_ref[...] - mu) * (inv * g_ref[...]) + b_ref[...]
    h = jnp.maximum(h, 0.0)
    if final:
        out_ref[...] = (jnp.dot(h, cw_ref[...],
                                preferred_element_type=jnp.float32)
                        + cb_ref[...])
    else:
        out_ref[...] = h


def _make_c2(final):
    grid = N // BR
    rb = lambda i: (i, 0)
    cb = lambda i: (0, 0)
    K = 3 * HID
    in_specs = [
        pl.BlockSpec((BR, K), rb),
        pl.BlockSpec((8, K), cb),
        pl.BlockSpec((1, K), cb),
        pl.BlockSpec((1, K), cb),
    ]
    if final:
        in_specs += [pl.BlockSpec((K, HID), cb), pl.BlockSpec((1, HID), cb)]
        out_w = HID
    else:
        out_w = K
    return pl.pallas_call(
        functools.partial(_c2_body, final),
        grid=(grid,),
        in_specs=in_specs,
        out_specs=pl.BlockSpec((BR, out_w), rb),
        out_shape=jax.ShapeDtypeStruct((N, out_w), jnp.float32),
    )


def _layer_aggregate(tbl2d, ncw, srcl, ldstl, cnt):
    sums, maxs, mins = [], [], []
    for c in range(ncw):
        s, m, n = _make_agg(ncw, c)(tbl2d, srcl, ldstl, cnt)
        sums.append(s.reshape(NPAD, W))
        maxs.append(m.reshape(NPAD, W))
        mins.append(n.reshape(NPAD, W))
    sm = jnp.concatenate(sums, axis=1)[:N]
    mx = jnp.concatenate(maxs, axis=1)[:N]
    mn = jnp.concatenate(mins, axis=1)[:N]
    return sm, mx, mn


def kernel(x, edge_index,
           Wl_0_0, bl_0_0, Wr_0_0,
           Wl_0_1, bl_0_1, Wr_0_1,
           Wl_0_2, bl_0_2, Wr_0_2,
           bn_g_0, bn_b_0,
           Wl_1_0, bl_1_0, Wr_1_0,
           Wl_1_1, bl_1_1, Wr_1_1,
           Wl_1_2, bl_1_2, Wr_1_2,
           bn_g_1, bn_b_1,
           clf_W, clf_b):
    srcl, ldstl, cnt = _bin_edges(edge_index.reshape(2 * E))
    srcl, ldstl, deg = _sort_bins(srcl, ldstl, cnt)

    sm0, mx0, mn0 = _layer_aggregate(x, D_IN // W, srcl, ldstl, cnt)
    degv = deg.reshape(NV, 160)[:, :R].reshape(NPAD, 1)[:N]
    c1 = _make_c1(D_IN)
    pre0, stat0 = c1(x, sm0, mx0, mn0, degv,
                     Wl_0_0, Wr_0_0, Wl_0_1, Wr_0_1, Wl_0_2, Wr_0_2,
                     bl_0_0.reshape(1, HID), bl_0_1.reshape(1, HID),
                     bl_0_2.reshape(1, HID))
    h1 = _make_c2(False)(pre0, stat0, bn_g_0.reshape(1, -1),
                         bn_b_0.reshape(1, -1))

    tbl1 = h1.reshape(N * (3 * HID // W), W)
    sm1, mx1, mn1 = _layer_aggregate(tbl1, 3 * HID // W, srcl, ldstl, cnt)
    c1b = _make_c1(3 * HID)
    pre1, stat1 = c1b(h1, sm1, mx1, mn1, degv,
                      Wl_1_0, Wr_1_0, Wl_1_1, Wr_1_1, Wl_1_2, Wr_1_2,
                      bl_1_0.reshape(1, HID), bl_1_1.reshape(1, HID),
                      bl_1_2.reshape(1, HID))
    clf_Wp = jnp.pad(clf_W, ((0, 0), (0, HID - clf_W.shape[1])))
    clf_bp = jnp.pad(clf_b, (0, HID - clf_b.shape[0])).reshape(1, HID)
    logits = _make_c2(True)(pre1, stat1, bn_g_1.reshape(1, -1),
                            bn_b_1.reshape(1, -1), clf_Wp, clf_bp)
    return logits[:, :clf_W.shape[1]]


# speedup vs baseline: 7.7833x; 1.3902x over previous
import functools

import jax
import jax.numpy as jnp
from jax import lax
from jax.experimental import pallas as pl
from jax.experimental.pallas import tpu as pltpu
from jax.experimental.pallas import tpu_sc as plsc

N = 10000
E = 320000
D_IN = 128
HID = 128
NC, NS = 2, 16
NW = NC * NS
NB = 2
NV = NW * NB
R = 157
NPAD = NV * R
W = 128
EBLK = 256
KSC = 3200
FLUSH = 4096
BUF = 8704
CAP = E + 2 * FLUSH
ACC = (R + 1) * W
NEG = float("-inf")
POS = float("inf")

_mesh = plsc.VectorSubcoreMesh(core_axis_name="c", subcore_axis_name="s")
_sc_params = pltpu.CompilerParams(needs_layout_passes=False)


def _wid():
    return lax.axis_index("s") * NC + lax.axis_index("c")


def _bin_body(edges, srcl, ldstl, cntl,
              s_scan, d_scan, s_buf0, d_buf0, s_buf1, d_buf1, cbuf):
    wid = _wid()
    base = wid * (NB * R)
    iota = lax.iota(jnp.int32, 16)
    sbufs = (s_buf0, s_buf1)
    dbufs = (d_buf0, d_buf1)

    def chunk(k, carry):
        t0, w0, t1, w1 = carry
        pltpu.sync_copy(edges.at[pl.ds(k * KSC, KSC)], s_scan)
        pltpu.sync_copy(edges.at[pl.ds(E + k * KSC, KSC)], d_scan)

        def inner(i, wps):
            w0, w1 = wps
            ss, dls, m0s, m1s, ps0s, ps1s = [], [], [], [], [], []
            for u in range(4):
                s = s_scan[pl.ds((i * 4 + u) * 16, 16)]
                d = d_scan[pl.ds((i * 4 + u) * 16, 16)]
                dl = d - base
                m0 = (dl >= 0) & (dl < R)
                m1 = (dl >= R) & (dl < 2 * R)
                ss.append(s)
                dls.append(dl)
                m0s.append(m0)
                m1s.append(m1)
                ps0s.append(plsc.cumsum(jnp.where(m0, 1, 0)))
                ps1s.append(plsc.cumsum(jnp.where(m1, 1, 0)))
            for u in range(4):
                plsc.store_scatter(s_buf0, [w0 + ps0s[u] - 1], ss[u],
                                   mask=m0s[u])
                plsc.store_scatter(d_buf0, [w0 + ps0s[u] - 1], dls[u],
                                   mask=m0s[u])
                plsc.store_scatter(s_buf1, [w1 + ps1s[u] - 1], ss[u],
                                   mask=m1s[u])
                plsc.store_scatter(d_buf1, [w1 + ps1s[u] - 1], dls[u] - R,
                                   mask=m1s[u])
                w0 = w0 + ps0s[u][15]
                w1 = w1 + ps1s[u][15]
            return w0, w1

        w0, w1 = lax.fori_loop(0, KSC // 64, inner, (w0, w1))

        totals = [t0, t1]
        wps = [w0, w1]
        for b in range(NB):
            v = wid * NB + b
            do_flush = wps[b] >= FLUSH
            tot = totals[b]
            sb, db = sbufs[b], dbufs[b]

            @pl.when(do_flush)
            def _(tot=tot, sb=sb, db=db, v=v):
                toff = pl.multiple_of(tot, 8)
                pltpu.sync_copy(sb.at[pl.ds(0, FLUSH)],
                                srcl.at[pl.ds(v * CAP + toff, FLUSH)])
                pltpu.sync_copy(db.at[pl.ds(0, FLUSH)],
                                ldstl.at[pl.ds(v * CAP + toff, FLUSH)])

                def shift(i, _):
                    sb[pl.ds(i * 16, 16)] = sb[pl.ds(FLUSH + i * 16, 16)]
                    db[pl.ds(i * 16, 16)] = db[pl.ds(FLUSH + i * 16, 16)]
                    return 0

                lax.fori_loop(0, (BUF - FLUSH) // 16, shift, 0)

            totals[b] = jnp.where(do_flush, tot + FLUSH, tot)
            wps[b] = jnp.where(do_flush, wps[b] - FLUSH, wps[b])
        return totals[0], wps[0], totals[1], wps[1]

    z = jnp.int32(0)
    t0, w0, t1, w1 = lax.fori_loop(0, E // KSC, chunk, (z, z, z, z))

    ones = jnp.full((16,), True)
    totals = [t0, t1]
    wps = [w0, w1]
    for b in range(NB):
        v = wid * NB + b
        sent_s = v * R + iota
        sent_d = jnp.full((16,), R, jnp.int32)
        sb, db = sbufs[b], dbufs[b]
        wp = wps[b]
        for j in range(EBLK // 16):
            plsc.store_scatter(sb, [wp + j * 16 + iota], sent_s, mask=ones)
            plsc.store_scatter(db, [wp + j * 16 + iota], sent_d, mask=ones)
        wp = ((wp + EBLK - 1) // EBLK) * EBLK
        tot = totals[b]
        toff = pl.multiple_of(tot, 8)

        @pl.when(wp > 0)
        def _(sb=sb, db=db, toff=toff, v=v):
            pltpu.sync_copy(sb.at[pl.ds(0, FLUSH)],
                            srcl.at[pl.ds(v * CAP + toff, FLUSH)])
            pltpu.sync_copy(db.at[pl.ds(0, FLUSH)],
                            ldstl.at[pl.ds(v * CAP + toff, FLUSH)])

        @pl.when(wp > FLUSH)
        def _(sb=sb, db=db, toff=toff, v=v):
            pltpu.sync_copy(sb.at[pl.ds(FLUSH, FLUSH)],
                            srcl.at[pl.ds(v * CAP + toff + FLUSH, FLUSH)])
            pltpu.sync_copy(db.at[pl.ds(FLUSH, FLUSH)],
                            ldstl.at[pl.ds(v * CAP + toff + FLUSH, FLUSH)])

        cbuf[...] = jnp.full((16,), tot + wp, jnp.int32)
        pltpu.sync_copy(cbuf, cntl.at[pl.ds(v * 16, 16)])


_bin_edges = functools.partial(
    pl.kernel,
    out_type=[
        jax.ShapeDtypeStruct((NV * CAP,), jnp.int32),
        jax.ShapeDtypeStruct((NV * CAP,), jnp.int32),
        jax.ShapeDtypeStruct((NV * 16,), jnp.int32),
    ],
    mesh=_mesh,
    scratch_types=[
        pltpu.VMEM((KSC,), jnp.int32),
        pltpu.VMEM((KSC,), jnp.int32),
        pltpu.VMEM((BUF,), jnp.int32),
        pltpu.VMEM((BUF,), jnp.int32),
        pltpu.VMEM((BUF,), jnp.int32),
        pltpu.VMEM((BUF,), jnp.int32),
        pltpu.VMEM((16,), jnp.int32),
    ],
    compiler_params=_sc_params,
)(_bin_body)


CAPS = 49152
SBLK = 4096


def _sort_body(srcl, ldstl, cntl, osrcl, ooffl, odeg,
               sblk, dblk, osrc, wo, histv, offv, degf, cntb):
    wid = _wid()
    zero_i = jnp.zeros((16,), jnp.int32)
    one0 = jnp.where(lax.iota(jnp.int32, 16) == 0, 1, 0)
    lane0 = lax.iota(jnp.int32, 16) == 0

    for b in range(NB):
        v = wid * NB + b
        for i in range(176 // 16):
            histv[pl.ds(i * 16, 16)] = zero_i

        pltpu.sync_copy(cntl.at[pl.ds(v * 16, 16)], cntb)
        cnt = cntb[pl.ds(0, 16)][0]
        nf = (cnt + SBLK - 1) // SBLK

        def hblk(bi, _):
            boff = pl.multiple_of(bi * SBLK, 8)
            pltpu.sync_copy(ldstl.at[pl.ds(v * CAP + boff, SBLK)], dblk)
            nin = jnp.minimum(cnt - bi * SBLK, SBLK)

            def hgrp(g, _):
                dvec = dblk[pl.ds(g * 16, 16)]
                for lane in range(16):
                    d = dvec[lane]
                    dsl = pl.ds(d, 16)
                    histv[dsl] = histv[dsl] + one0
                return 0

            lax.fori_loop(0, nin // 16, hgrp, 0)
            return 0

        lax.fori_loop(0, nf, hblk, 0)

        for i in range(160 // 16):
            degf[pl.ds(i * 16, 16)] = histv[pl.ds(i * 16, 16)].astype(
                jnp.float32)
        pltpu.sync_copy(degf, odeg.at[pl.ds(v * 160, 160)])

        carry = jnp.int32(0)
        for i in range(176 // 16):
            hv = histv[pl.ds(i * 16, 16)]
            ps = plsc.cumsum(hv)
            offv[pl.ds(i * 16, 16)] = ps - hv + carry
            carry = carry + ps[15]

        pltpu.sync_copy(offv.at[pl.ds(0, 160)], ooffl.at[pl.ds(v * 160, 160)])

        npass = (cnt + CAPS - 1) // CAPS

        def ppass(w, _):
            w0 = pl.multiple_of(w * CAPS, 8)
            for i in range(176 // 16):
                wo[pl.ds(i * 16, 16)] = offv[pl.ds(i * 16, 16)]

            def pblk(bi, _):
                boff = pl.multiple_of(bi * SBLK, 8)
                pltpu.sync_copy(srcl.at[pl.ds(v * CAP + boff, SBLK)], sblk)
                pltpu.sync_copy(ldstl.at[pl.ds(v * CAP + boff, SBLK)], dblk)
                nin = jnp.minimum(cnt - bi * SBLK, SBLK)

                def pgrp(g, _):
                    dvec = dblk[pl.ds(g * 16, 16)]
                    svec = sblk[pl.ds(g * 16, 16)]
                    for lane in range(16):
                        d = dvec[lane]
                        s = svec[lane]
                        dsl = pl.ds(d, 16)
                        ov = wo[dsl]
                        p = ov[0]
                        wo[dsl] = ov + one0
                        pw = p - w0
                        inwin = (pw >= 0) & (pw < CAPS)
                        mk = lane0 & jnp.full((16,), inwin)
                        pv = jnp.full((16,), pw, jnp.int32)
                        plsc.store_scatter(osrc, [pv],
                                           jnp.full((16,), s, jnp.int32),
                                           mask=mk)
                    return 0

                lax.fori_loop(0, nin // 16, pgrp, 0)
                return 0

            lax.fori_loop(0, nf, pblk, 0)

            nw = (jnp.minimum(cnt - w0, CAPS) + SBLK - 1) // SBLK

            def wblk(bi, _):
                boff = pl.multiple_of(bi * SBLK, 8)
                pltpu.sync_copy(osrc.at[pl.ds(boff, SBLK)],
                                osrcl.at[pl.ds(v * CAP + w0 + boff, SBLK)])
                return 0

            lax.fori_loop(0, nw, wblk, 0)
            return 0

        lax.fori_loop(0, npass, ppass, 0)


_sort_bins = functools.partial(
    pl.kernel,
    out_type=[
        jax.ShapeDtypeStruct((NV * CAP,), jnp.int32),
        jax.ShapeDtypeStruct((NV * 160,), jnp.int32),
        jax.ShapeDtypeStruct((NV * 160,), jnp.float32),
    ],
    mesh=_mesh,
    scratch_types=[
        pltpu.VMEM((SBLK,), jnp.int32),
        pltpu.VMEM((SBLK,), jnp.int32),
        pltpu.VMEM((CAPS,), jnp.int32),
        pltpu.VMEM((176,), jnp.int32),
        pltpu.VMEM((176,), jnp.int32),
        pltpu.VMEM((176,), jnp.int32),
        pltpu.VMEM((160,), jnp.float32),
        pltpu.VMEM((16,), jnp.int32),
    ],
    compiler_params=_sc_params,
)(_sort_body)


def _agg_body(ncw, c, tbl, srcl, ooffl, cntl, *refs):
    (osum, omax, omin, srcb0, srcb1, idx0, idx1, gbuf0, gbuf1,
     accs, accm, accn, offb, cntb, sem0, sem1) = refs
    wid = _wid()

    zero = jnp.zeros((16,), jnp.float32)
    negs = zero + NEG
    poss = zero + POS
    NJ = W // 16
    bufs = ((srcb0, idx0, gbuf0, sem0), (srcb1, idx1, gbuf1, sem1))

    for b in range(NB):
        v = wid * NB + b

        def initr(i, _):
            sl = pl.ds(i * 16, 16)
            accs[sl] = zero
            accm[sl] = negs
            accn[sl] = poss
            return 0

        lax.fori_loop(0, ACC // 16, initr, 0)

        pltpu.sync_copy(cntl.at[pl.ds(v * 16, 16)], cntb)
        pltpu.sync_copy(ooffl.at[pl.ds(v * 160, 160)], offb.at[pl.ds(0, 160)])
        cnt = cntb[pl.ds(0, 16)][0]
        nblk = cnt // EBLK

        def issue(bi, p):
            srcb, idx2, gbuf, sem = bufs[p]
            boff = pl.multiple_of(bi * EBLK, 8)
            pltpu.sync_copy(srcl.at[pl.ds(v * CAP + boff, EBLK)], srcb)
            for i in range(EBLK // 16):
                vv = srcb[pl.ds(i * 16, 16)] * ncw + c
                idx2[i // 8, pl.ds((i % 8) * 16, 16)] = vv
            for j in range(EBLK // 128):
                pltpu.async_copy(tbl.at[idx2.at[j]],
                                 gbuf.at[pl.ds(j * 128, 128)], sem)

        def wait_g(p):
            srcb, idx2, gbuf, sem = bufs[p]
            for j in range(EBLK // 128):
                pltpu.make_async_copy(tbl.at[idx2.at[j]],
                                      gbuf.at[pl.ds(j * 128, 128)],
                                      sem).wait()

        def compute(bi, p, carry):
            gbuf = bufs[p][2]
            e0 = bi * EBLK

            def seg_cond(st):
                return st[1] < EBLK

            def seg_body(st):
                r = st[0]
                pos = st[1]
                regs = list(st[2:])
                rend = offb[pl.ds(r + 1, 16)][0] - e0
                send = jnp.minimum(rend, EBLK)

                def acc_e(el, regs2):
                    regs2 = list(regs2)
                    for j in range(NJ):
                        rr = gbuf[el, pl.ds(j * 16, 16)]
                        regs2[j] = regs2[j] + rr
                        regs2[NJ + j] = jnp.maximum(regs2[NJ + j], rr)
                        regs2[2 * NJ + j] = jnp.minimum(regs2[2 * NJ + j], rr)
                    return tuple(regs2)

                regs = list(lax.fori_loop(pos, send, acc_e, tuple(regs)))
                fin = rend <= EBLK

                def flush(args, rr=r):
                    for j in range(NJ):
                        sl = pl.ds(rr * W + j * 16, 16)
                        accs[sl] = accs[sl] + args[j]
                        accm[sl] = jnp.maximum(accm[sl], args[NJ + j])
                        accn[sl] = jnp.minimum(accn[sl], args[2 * NJ + j])
                    return ([zero] * NJ) + ([negs] * NJ) + ([poss] * NJ)

                regs = lax.cond(fin, flush, lambda a: list(a), tuple(regs))
                r = jnp.where(fin, r + 1, r)
                return (r, send, *regs)

            st = lax.while_loop(seg_cond, seg_body,
                                (carry[0], jnp.int32(0), *carry[1:]))
            return (st[0], *st[2:])

        issue(0, 0)

        def pair(i, carry):
            b0 = 2 * i
            b1 = 2 * i + 1
            wait_g(0)

            @pl.when(b1 < nblk)
            def _():
                issue(b1, 1)

            carry = compute(b0, 0, carry)

            def second(cc):
                @pl.when(b1 + 1 < nblk)
                def _():
                    issue(b1 + 1, 0)

                wait_g(1)
                return compute(b1, 1, cc)

            return lax.cond(b1 < nblk, second, lambda cc: cc, carry)

        init = (jnp.int32(0),) + tuple([zero] * NJ + [negs] * NJ + [poss] * NJ)
        lax.fori_loop(0, (nblk + 1) // 2, pair, init)

        pltpu.sync_copy(accs.at[pl.ds(0, R * W)],
                        osum.at[pl.ds(v * R * W, R * W)])
        pltpu.sync_copy(accm.at[pl.ds(0, R * W)],
                        omax.at[pl.ds(v * R * W, R * W)])
        pltpu.sync_copy(accn.at[pl.ds(0, R * W)],
                        omin.at[pl.ds(v * R * W, R * W)])


def _make_agg(ncw, c):
    outs = [jax.ShapeDtypeStruct((NPAD * W,), jnp.float32)] * 3
    scratch = [
        pltpu.VMEM((EBLK,), jnp.int32),
        pltpu.VMEM((EBLK,), jnp.int32),
        pltpu.VMEM((EBLK // 128, 128), jnp.int32),
        pltpu.VMEM((EBLK // 128, 128), jnp.int32),
        pltpu.VMEM((EBLK, W), jnp.float32),
        pltpu.VMEM((EBLK, W), jnp.float32),
        pltpu.VMEM((ACC,), jnp.float32),
        pltpu.VMEM((ACC,), jnp.float32),
        pltpu.VMEM((ACC,), jnp.float32),
        pltpu.VMEM((176,), jnp.int32),
        pltpu.VMEM((16,), jnp.int32),
        pltpu.SemaphoreType.DMA,
        pltpu.SemaphoreType.DMA,
    ]
    return pl.kernel(
        functools.partial(_agg_body, ncw, c),
        out_type=outs,
        mesh=_mesh,
        scratch_types=scratch,
        compiler_params=_sc_params,
    )


BR = 2000


def _c1_body(h_ref, sm_ref, mx_ref, mn_ref, dg_ref,
             wl0, wr0, wl1, wr1, wl2, wr2, bl0, bl1, bl2,
             opre, ostat):
    i = pl.program_id(0)
    deg = dg_ref[...]
    degc = jnp.maximum(deg, 1.0)
    emp = deg <= 0.0
    h = h_ref[...]
    mean = sm_ref[...] / degc
    mxv = jnp.where(emp, 0.0, mx_ref[...])
    mnv = jnp.where(emp, 0.0, mn_ref[...])
    parts = []
    for agg, Wl, bl, Wr in ((mean, wl0, bl0, wr0),
                            (mxv, wl1, bl1, wr1),
                            (mnv, wl2, bl2, wr2)):
        parts.append(
            jnp.dot(agg, Wl[...], preferred_element_type=jnp.float32)
            + bl[...]
            + jnp.dot(h, Wr[...], preferred_element_type=jnp.float32))
    pre = jnp.concatenate(parts, axis=1)
    opre[...] = pre

    @pl.when(i == 0)
    def _():
        ostat[...] = jnp.zeros_like(ostat)

    s0 = jnp.sum(pre, axis=0)[None, :]
    s1 = jnp.sum(pre * pre, axis=0)[None, :]
    pad = jnp.zeros((6, pre.shape[1]), jnp.float32)
    ostat[...] = ostat[...] + jnp.concatenate([s0, s1, pad], axis=0)


def _make_c1(K):
    grid = N // BR
    rb = lambda i: (i, 0)
    cb = lambda i: (0, 0)
    return pl.pallas_call(
        _c1_body,
        grid=(grid,),
        in_specs=[
            pl.BlockSpec((BR, K), rb),
            pl.BlockSpec((BR, K), rb),
            pl.BlockSpec((BR, K), rb),
            pl.BlockSpec((BR, K), rb),
            pl.BlockSpec((BR, 1), rb),
        ] + [pl.BlockSpec((K, HID), cb)] * 6 + [pl.BlockSpec((1, HID), cb)] * 3,
        out_specs=[
            pl.BlockSpec((BR, 3 * HID), rb),
            pl.BlockSpec((8, 3 * HID), cb),
        ],
        out_shape=[
            jax.ShapeDtypeStruct((N, 3 * HID), jnp.float32),
            jax.ShapeDtypeStruct((8, 3 * HID), jnp.float32),
        ],
    )


def _c2_body(final, pre_ref, stat_ref, g_ref, b_ref, *rest):
    if final:
        cw_ref, cb_ref, out_ref = rest
    else:
        (out_ref,) = rest
    stat = stat_ref[...]
    mu = stat[0:1, :] / jnp.float32(N)
    var = stat[1:2, :] / jnp.float32(N) - mu * mu
    inv = lax.rsqrt(var + 1e-5)
    h = (pre_ref[...] - mu) * (inv * g_ref[...]) + b_ref[...]
    h = jnp.maximum(h, 0.0)
    if final:
        out_ref[...] = (jnp.dot(h, cw_ref[...],
                                preferred_element_type=jnp.float32)
                        + cb_ref[...])
    else:
        out_ref[...] = h


def _make_c2(final):
    grid = N // BR
    rb = lambda i: (i, 0)
    cb = lambda i: (0, 0)
    K = 3 * HID
    in_specs = [
        pl.BlockSpec((BR, K), rb),
        pl.BlockSpec((8, K), cb),
        pl.BlockSpec((1, K), cb),
        pl.BlockSpec((1, K), cb),
    ]
    if final:
        in_specs += [pl.BlockSpec((K, HID), cb), pl.BlockSpec((1, HID), cb)]
        out_w = HID
    else:
        out_w = K
    return pl.pallas_call(
        functools.partial(_c2_body, final),
        grid=(grid,),
        in_specs=in_specs,
        out_specs=pl.BlockSpec((BR, out_w), rb),
        out_shape=jax.ShapeDtypeStruct((N, out_w), jnp.float32),
    )


def _layer_aggregate(tbl2d, ncw, srcl, ldstl, cnt):
    sums, maxs, mins = [], [], []
    for c in range(ncw):
        s, m, n = _make_agg(ncw, c)(tbl2d, srcl, ldstl, cnt)
        sums.append(s.reshape(NPAD, W))
        maxs.append(m.reshape(NPAD, W))
        mins.append(n.reshape(NPAD, W))
    sm = jnp.concatenate(sums, axis=1)[:N]
    mx = jnp.concatenate(maxs, axis=1)[:N]
    mn = jnp.concatenate(mins, axis=1)[:N]
    return sm, mx, mn


def kernel(x, edge_index,
           Wl_0_0, bl_0_0, Wr_0_0,
           Wl_0_1, bl_0_1, Wr_0_1,
           Wl_0_2, bl_0_2, Wr_0_2,
           bn_g_0, bn_b_0,
           Wl_1_0, bl_1_0, Wr_1_0,
           Wl_1_1, bl_1_1, Wr_1_1,
           Wl_1_2, bl_1_2, Wr_1_2,
           bn_g_1, bn_b_1,
           clf_W, clf_b):
    srcl, ldstl, cnt = _bin_edges(edge_index.reshape(2 * E))
    srcl, ldstl, deg = _sort_bins(srcl, ldstl, cnt)

    sm0, mx0, mn0 = _layer_aggregate(x, D_IN // W, srcl, ldstl, cnt)
    degv = deg.reshape(NV, 160)[:, :R].reshape(NPAD, 1)[:N]
    c1 = _make_c1(D_IN)
    pre0, stat0 = c1(x, sm0, mx0, mn0, degv,
                     Wl_0_0, Wr_0_0, Wl_0_1, Wr_0_1, Wl_0_2, Wr_0_2,
                     bl_0_0.reshape(1, HID), bl_0_1.reshape(1, HID),
                     bl_0_2.reshape(1, HID))
    h1 = _make_c2(False)(pre0, stat0, bn_g_0.reshape(1, -1),
                         bn_b_0.reshape(1, -1))

    tbl1 = h1.reshape(N * (3 * HID // W), W)
    sm1, mx1, mn1 = _layer_aggregate(tbl1, 3 * HID // W, srcl, ldstl, cnt)
    c1b = _make_c1(3 * HID)
    pre1, stat1 = c1b(h1, sm1, mx1, mn1, degv,
                      Wl_1_0, Wr_1_0, Wl_1_1, Wr_1_1, Wl_1_2, Wr_1_2,
                      bl_1_0.reshape(1, HID), bl_1_1.reshape(1, HID),
                      bl_1_2.reshape(1, HID))
    clf_Wp = jnp.pad(clf_W, ((0, 0), (0, HID - clf_W.shape[1])))
    clf_bp = jnp.pad(clf_b, (0, HID - clf_b.shape[0])).reshape(1, HID)
    logits = _make_c2(True)(pre1, stat1, bn_g_1.reshape(1, -1),
                            bn_b_1.reshape(1, -1), clf_Wp, clf_bp)
    return logits[:, :clf_W.shape[1]]


# speedup vs baseline: 8.0812x; 1.0383x over previous
import functools

import jax
import jax.numpy as jnp
from jax import lax
from jax.experimental import pallas as pl
from jax.experimental.pallas import tpu as pltpu
from jax.experimental.pallas import tpu_sc as plsc

N = 10000
E = 320000
D_IN = 128
HID = 128
NC, NS = 2, 16
NW = NC * NS
NB = 2
NV = NW * NB
R = 157
NPAD = NV * R
W = 128
EBLK = 256
KSC = 3200
FLUSH = 4096
BUF = 8704
CAP = E + 2 * FLUSH
ACC = (R + 1) * W
NEG = float("-inf")
POS = float("inf")

_mesh = plsc.VectorSubcoreMesh(core_axis_name="c", subcore_axis_name="s")
_sc_params = pltpu.CompilerParams(needs_layout_passes=False)


def _wid():
    return lax.axis_index("s") * NC + lax.axis_index("c")


def _bin_body(edges, srcl, ldstl, cntl,
              s_scan, d_scan, s_buf0, d_buf0, s_buf1, d_buf1, cbuf):
    wid = _wid()
    base = wid * (NB * R)
    iota = lax.iota(jnp.int32, 16)
    sbufs = (s_buf0, s_buf1)
    dbufs = (d_buf0, d_buf1)

    def chunk(k, carry):
        t0, w0, t1, w1 = carry
        pltpu.sync_copy(edges.at[pl.ds(k * KSC, KSC)], s_scan)
        pltpu.sync_copy(edges.at[pl.ds(E + k * KSC, KSC)], d_scan)

        def inner(i, wps):
            w0, w1 = wps
            ss, dls, m0s, m1s, ps0s, ps1s = [], [], [], [], [], []
            for u in range(4):
                s = s_scan[pl.ds((i * 4 + u) * 16, 16)]
                d = d_scan[pl.ds((i * 4 + u) * 16, 16)]
                dl = d - base
                m0 = (dl >= 0) & (dl < R)
                m1 = (dl >= R) & (dl < 2 * R)
                ss.append(s)
                dls.append(dl)
                m0s.append(m0)
                m1s.append(m1)
                ps0s.append(plsc.cumsum(jnp.where(m0, 1, 0)))
                ps1s.append(plsc.cumsum(jnp.where(m1, 1, 0)))
            for u in range(4):
                plsc.store_scatter(s_buf0, [w0 + ps0s[u] - 1], ss[u],
                                   mask=m0s[u])
                plsc.store_scatter(d_buf0, [w0 + ps0s[u] - 1], dls[u],
                                   mask=m0s[u])
                plsc.store_scatter(s_buf1, [w1 + ps1s[u] - 1], ss[u],
                                   mask=m1s[u])
                plsc.store_scatter(d_buf1, [w1 + ps1s[u] - 1], dls[u] - R,
                                   mask=m1s[u])
                w0 = w0 + ps0s[u][15]
                w1 = w1 + ps1s[u][15]
            return w0, w1

        w0, w1 = lax.fori_loop(0, KSC // 64, inner, (w0, w1))

        totals = [t0, t1]
        wps = [w0, w1]
        for b in range(NB):
            v = wid * NB + b
            do_flush = wps[b] >= FLUSH
            tot = totals[b]
            sb, db = sbufs[b], dbufs[b]

            @pl.when(do_flush)
            def _(tot=tot, sb=sb, db=db, v=v):
                toff = pl.multiple_of(tot, 8)
                pltpu.sync_copy(sb.at[pl.ds(0, FLUSH)],
                                srcl.at[pl.ds(v * CAP + toff, FLUSH)])
                pltpu.sync_copy(db.at[pl.ds(0, FLUSH)],
                                ldstl.at[pl.ds(v * CAP + toff, FLUSH)])

                def shift(i, _):
                    sb[pl.ds(i * 16, 16)] = sb[pl.ds(FLUSH + i * 16, 16)]
                    db[pl.ds(i * 16, 16)] = db[pl.ds(FLUSH + i * 16, 16)]
                    return 0

                lax.fori_loop(0, (BUF - FLUSH) // 16, shift, 0)

            totals[b] = jnp.where(do_flush, tot + FLUSH, tot)
            wps[b] = jnp.where(do_flush, wps[b] - FLUSH, wps[b])
        return totals[0], wps[0], totals[1], wps[1]

    z = jnp.int32(0)
    t0, w0, t1, w1 = lax.fori_loop(0, E // KSC, chunk, (z, z, z, z))

    ones = jnp.full((16,), True)
    totals = [t0, t1]
    wps = [w0, w1]
    for b in range(NB):
        v = wid * NB + b
        sent_s = v * R + iota
        sent_d = jnp.full((16,), R, jnp.int32)
        sb, db = sbufs[b], dbufs[b]
        wp = wps[b]
        for j in range(EBLK // 16):
            plsc.store_scatter(sb, [wp + j * 16 + iota], sent_s, mask=ones)
            plsc.store_scatter(db, [wp + j * 16 + iota], sent_d, mask=ones)
        wp = ((wp + EBLK - 1) // EBLK) * EBLK
        tot = totals[b]
        toff = pl.multiple_of(tot, 8)

        @pl.when(wp > 0)
        def _(sb=sb, db=db, toff=toff, v=v):
            pltpu.sync_copy(sb.at[pl.ds(0, FLUSH)],
                            srcl.at[pl.ds(v * CAP + toff, FLUSH)])
            pltpu.sync_copy(db.at[pl.ds(0, FLUSH)],
                            ldstl.at[pl.ds(v * CAP + toff, FLUSH)])

        @pl.when(wp > FLUSH)
        def _(sb=sb, db=db, toff=toff, v=v):
            pltpu.sync_copy(sb.at[pl.ds(FLUSH, FLUSH)],
                            srcl.at[pl.ds(v * CAP + toff + FLUSH, FLUSH)])
            pltpu.sync_copy(db.at[pl.ds(FLUSH, FLUSH)],
                            ldstl.at[pl.ds(v * CAP + toff + FLUSH, FLUSH)])

        cbuf[...] = jnp.full((16,), tot + wp, jnp.int32)
        pltpu.sync_copy(cbuf, cntl.at[pl.ds(v * 16, 16)])


_bin_edges = functools.partial(
    pl.kernel,
    out_type=[
        jax.ShapeDtypeStruct((NV * CAP,), jnp.int32),
        jax.ShapeDtypeStruct((NV * CAP,), jnp.int32),
        jax.ShapeDtypeStruct((NV * 16,), jnp.int32),
    ],
    mesh=_mesh,
    scratch_types=[
        pltpu.VMEM((KSC,), jnp.int32),
        pltpu.VMEM((KSC,), jnp.int32),
        pltpu.VMEM((BUF,), jnp.int32),
        pltpu.VMEM((BUF,), jnp.int32),
        pltpu.VMEM((BUF,), jnp.int32),
        pltpu.VMEM((BUF,), jnp.int32),
        pltpu.VMEM((16,), jnp.int32),
    ],
    compiler_params=_sc_params,
)(_bin_body)


CAPS = 49152
SBLK = 4096


def _sort_body(srcl, ldstl, cntl, osrcl, ooffl, odeg,
               sblk, dblk, osrc, wo, histv, offv, degf, cntb):
    wid = _wid()
    zero_i = jnp.zeros((16,), jnp.int32)
    one0 = jnp.where(lax.iota(jnp.int32, 16) == 0, 1, 0)
    lane0 = lax.iota(jnp.int32, 16) == 0

    for b in range(NB):
        v = wid * NB + b
        for i in range(176 // 16):
            histv[pl.ds(i * 16, 16)] = zero_i

        pltpu.sync_copy(cntl.at[pl.ds(v * 16, 16)], cntb)
        cnt = cntb[pl.ds(0, 16)][0]
        nf = (cnt + SBLK - 1) // SBLK

        def hblk(bi, _):
            boff = pl.multiple_of(bi * SBLK, 8)
            pltpu.sync_copy(ldstl.at[pl.ds(v * CAP + boff, SBLK)], dblk)
            nin = jnp.minimum(cnt - bi * SBLK, SBLK)

            ones16 = jnp.full((16,), 1, jnp.int32)
            tmask = jnp.full((16,), True)

            def hgrp(g, _):
                dvec = dblk[pl.ds(g * 16, 16)]
                plsc.addupdate_scatter(histv, [dvec], ones16, mask=tmask)
                return 0

            lax.fori_loop(0, nin // 16, hgrp, 0)
            return 0

        lax.fori_loop(0, nf, hblk, 0)

        for i in range(160 // 16):
            degf[pl.ds(i * 16, 16)] = histv[pl.ds(i * 16, 16)].astype(
                jnp.float32)
        pltpu.sync_copy(degf, odeg.at[pl.ds(v * 160, 160)])

        carry = jnp.int32(0)
        for i in range(176 // 16):
            hv = histv[pl.ds(i * 16, 16)]
            ps = plsc.cumsum(hv)
            offv[pl.ds(i * 16, 16)] = ps - hv + carry
            carry = carry + ps[15]

        pltpu.sync_copy(offv.at[pl.ds(0, 160)], ooffl.at[pl.ds(v * 160, 160)])

        npass = (cnt + CAPS - 1) // CAPS

        def ppass(w, _):
            w0 = pl.multiple_of(w * CAPS, 8)
            for i in range(176 // 16):
                wo[pl.ds(i * 16, 16)] = offv[pl.ds(i * 16, 16)]

            def pblk(bi, _):
                boff = pl.multiple_of(bi * SBLK, 8)
                pltpu.sync_copy(srcl.at[pl.ds(v * CAP + boff, SBLK)], sblk)
                pltpu.sync_copy(ldstl.at[pl.ds(v * CAP + boff, SBLK)], dblk)
                nin = jnp.minimum(cnt - bi * SBLK, SBLK)

                def pgrp(g, _):
                    dvec = dblk[pl.ds(g * 16, 16)]
                    svec = sblk[pl.ds(g * 16, 16)]
                    for lane in range(16):
                        d = dvec[lane]
                        s = svec[lane]
                        dsl = pl.ds(d, 16)
                        ov = wo[dsl]
                        p = ov[0]
                        wo[dsl] = ov + one0
                        pw = p - w0
                        inwin = (pw >= 0) & (pw < CAPS)
                        mk = lane0 & jnp.full((16,), inwin)
                        pv = jnp.full((16,), pw, jnp.int32)
                        plsc.store_scatter(osrc, [pv],
                                           jnp.full((16,), s, jnp.int32),
                                           mask=mk)
                    return 0

                lax.fori_loop(0, nin // 16, pgrp, 0)
                return 0

            lax.fori_loop(0, nf, pblk, 0)

            nw = (jnp.minimum(cnt - w0, CAPS) + SBLK - 1) // SBLK

            def wblk(bi, _):
                boff = pl.multiple_of(bi * SBLK, 8)
                pltpu.sync_copy(osrc.at[pl.ds(boff, SBLK)],
                                osrcl.at[pl.ds(v * CAP + w0 + boff, SBLK)])
                return 0

            lax.fori_loop(0, nw, wblk, 0)
            return 0

        lax.fori_loop(0, npass, ppass, 0)


_sort_bins = functools.partial(
    pl.kernel,
    out_type=[
        jax.ShapeDtypeStruct((NV * CAP,), jnp.int32),
        jax.ShapeDtypeStruct((NV * 160,), jnp.int32),
        jax.ShapeDtypeStruct((NV * 160,), jnp.float32),
    ],
    mesh=_mesh,
    scratch_types=[
        pltpu.VMEM((SBLK,), jnp.int32),
        pltpu.VMEM((SBLK,), jnp.int32),
        pltpu.VMEM((CAPS,), jnp.int32),
        pltpu.VMEM((176,), jnp.int32),
        pltpu.VMEM((176,), jnp.int32),
        pltpu.VMEM((176,), jnp.int32),
        pltpu.VMEM((160,), jnp.float32),
        pltpu.VMEM((16,), jnp.int32),
    ],
    compiler_params=_sc_params,
)(_sort_body)


def _agg_body(ncw, c, tbl, srcl, ooffl, cntl, *refs):
    (osum, omax, omin, srcb0, srcb1, idx0, idx1, gbuf0, gbuf1,
     accs, accm, accn, offb, cntb, sem0, sem1) = refs
    wid = _wid()

    zero = jnp.zeros((16,), jnp.float32)
    negs = zero + NEG
    poss = zero + POS
    NJ = W // 16
    bufs = ((srcb0, idx0, gbuf0, sem0), (srcb1, idx1, gbuf1, sem1))

    for b in range(NB):
        v = wid * NB + b

        def initr(i, _):
            sl = pl.ds(i * 16, 16)
            accs[sl] = zero
            accm[sl] = negs
            accn[sl] = poss
            return 0

        lax.fori_loop(0, ACC // 16, initr, 0)

        pltpu.sync_copy(cntl.at[pl.ds(v * 16, 16)], cntb)
        pltpu.sync_copy(ooffl.at[pl.ds(v * 160, 160)], offb.at[pl.ds(0, 160)])
        cnt = cntb[pl.ds(0, 16)][0]
        nblk = cnt // EBLK

        def issue(bi, p):
            srcb, idx2, gbuf, sem = bufs[p]
            boff = pl.multiple_of(bi * EBLK, 8)
            pltpu.sync_copy(srcl.at[pl.ds(v * CAP + boff, EBLK)], srcb)
            for i in range(EBLK // 16):
                vv = srcb[pl.ds(i * 16, 16)] * ncw + c
                idx2[i // 8, pl.ds((i % 8) * 16, 16)] = vv
            for j in range(EBLK // 128):
                pltpu.async_copy(tbl.at[idx2.at[j]],
                                 gbuf.at[pl.ds(j * 128, 128)], sem)

        def wait_g(p):
            srcb, idx2, gbuf, sem = bufs[p]
            for j in range(EBLK // 128):
                pltpu.make_async_copy(tbl.at[idx2.at[j]],
                                      gbuf.at[pl.ds(j * 128, 128)],
                                      sem).wait()

        def compute(bi, p, carry):
            gbuf = bufs[p][2]
            e0 = bi * EBLK

            def seg_cond(st):
                return st[1] < EBLK

            def seg_body(st):
                r = st[0]
                pos = st[1]
                regs = list(st[2:])
                rend = offb[pl.ds(r + 1, 16)][0] - e0
                send = jnp.minimum(rend, EBLK)

                def acc_e(el, regs2):
                    regs2 = list(regs2)
                    for j in range(NJ):
                        rr = gbuf[el, pl.ds(j * 16, 16)]
                        regs2[j] = regs2[j] + rr
                        regs2[NJ + j] = jnp.maximum(regs2[NJ + j], rr)
                        regs2[2 * NJ + j] = jnp.minimum(regs2[2 * NJ + j], rr)
                    return tuple(regs2)

                regs = list(lax.fori_loop(pos, send, acc_e, tuple(regs)))
                fin = rend <= EBLK

                def flush(args, rr=r):
                    for j in range(NJ):
                        sl = pl.ds(rr * W + j * 16, 16)
                        accs[sl] = accs[sl] + args[j]
                        accm[sl] = jnp.maximum(accm[sl], args[NJ + j])
                        accn[sl] = jnp.minimum(accn[sl], args[2 * NJ + j])
                    return ([zero] * NJ) + ([negs] * NJ) + ([poss] * NJ)

                regs = lax.cond(fin, flush, lambda a: list(a), tuple(regs))
                r = jnp.where(fin, r + 1, r)
                return (r, send, *regs)

            st = lax.while_loop(seg_cond, seg_body,
                                (carry[0], jnp.int32(0), *carry[1:]))
            return (st[0], *st[2:])

        issue(0, 0)

        def pair(i, carry):
            b0 = 2 * i
            b1 = 2 * i + 1
            wait_g(0)

            @pl.when(b1 < nblk)
            def _():
                issue(b1, 1)

            carry = compute(b0, 0, carry)

            def second(cc):
                @pl.when(b1 + 1 < nblk)
                def _():
                    issue(b1 + 1, 0)

                wait_g(1)
                return compute(b1, 1, cc)

            return lax.cond(b1 < nblk, second, lambda cc: cc, carry)

        init = (jnp.int32(0),) + tuple([zero] * NJ + [negs] * NJ + [poss] * NJ)
        lax.fori_loop(0, (nblk + 1) // 2, pair, init)

        pltpu.sync_copy(accs.at[pl.ds(0, R * W)],
                        osum.at[pl.ds(v * R * W, R * W)])
        pltpu.sync_copy(accm.at[pl.ds(0, R * W)],
                        omax.at[pl.ds(v * R * W, R * W)])
        pltpu.sync_copy(accn.at[pl.ds(0, R * W)],
                        omin.at[pl.ds(v * R * W, R * W)])


def _make_agg(ncw, c):
    outs = [jax.ShapeDtypeStruct((NPAD * W,), jnp.float32)] * 3
    scratch = [
        pltpu.VMEM((EBLK,), jnp.int32),
        pltpu.VMEM((EBLK,), jnp.int32),
        pltpu.VMEM((EBLK // 128, 128), jnp.int32),
        pltpu.VMEM((EBLK // 128, 128), jnp.int32),
        pltpu.VMEM((EBLK, W), jnp.float32),
        pltpu.VMEM((EBLK, W), jnp.float32),
        pltpu.VMEM((ACC,), jnp.float32),
        pltpu.VMEM((ACC,), jnp.float32),
        pltpu.VMEM((ACC,), jnp.float32),
        pltpu.VMEM((176,), jnp.int32),
        pltpu.VMEM((16,), jnp.int32),
        pltpu.SemaphoreType.DMA,
        pltpu.SemaphoreType.DMA,
    ]
    return pl.kernel(
        functools.partial(_agg_body, ncw, c),
        out_type=outs,
        mesh=_mesh,
        scratch_types=scratch,
        compiler_params=_sc_params,
    )


BR = 2000


def _c1_body(h_ref, sm_ref, mx_ref, mn_ref, dg_ref,
             wl0, wr0, wl1, wr1, wl2, wr2, bl0, bl1, bl2,
             opre, ostat):
    i = pl.program_id(0)
    deg = dg_ref[...]
    degc = jnp.maximum(deg, 1.0)
    emp = deg <= 0.0
    h = h_ref[...]
    mean = sm_ref[...] / degc
    mxv = jnp.where(emp, 0.0, mx_ref[...])
    mnv = jnp.where(emp, 0.0, mn_ref[...])
    parts = []
    for agg, Wl, bl, Wr in ((mean, wl0, bl0, wr0),
                            (mxv, wl1, bl1, wr1),
                            (mnv, wl2, bl2, wr2)):
        parts.append(
            jnp.dot(agg, Wl[...], preferred_element_type=jnp.float32)
            + bl[...]
            + jnp.dot(h, Wr[...], preferred_element_type=jnp.float32))
    pre = jnp.concatenate(parts, axis=1)
    opre[...] = pre

    @pl.when(i == 0)
    def _():
        ostat[...] = jnp.zeros_like(ostat)

    s0 = jnp.sum(pre, axis=0)[None, :]
    s1 = jnp.sum(pre * pre, axis=0)[None, :]
    pad = jnp.zeros((6, pre.shape[1]), jnp.float32)
    ostat[...] = ostat[...] + jnp.concatenate([s0, s1, pad], axis=0)


def _make_c1(K):
    grid = N // BR
    rb = lambda i: (i, 0)
    cb = lambda i: (0, 0)
    return pl.pallas_call(
        _c1_body,
        grid=(grid,),
        in_specs=[
            pl.BlockSpec((BR, K), rb),
            pl.BlockSpec((BR, K), rb),
            pl.BlockSpec((BR, K), rb),
            pl.BlockSpec((BR, K), rb),
            pl.BlockSpec((BR, 1), rb),
        ] + [pl.BlockSpec((K, HID), cb)] * 6 + [pl.BlockSpec((1, HID), cb)] * 3,
        out_specs=[
            pl.BlockSpec((BR, 3 * HID), rb),
            pl.BlockSpec((8, 3 * HID), cb),
        ],
        out_shape=[
            jax.ShapeDtypeStruct((N, 3 * HID), jnp.float32),
            jax.ShapeDtypeStruct((8, 3 * HID), jnp.float32),
        ],
    )


def _c2_body(final, pre_ref, stat_ref, g_ref, b_ref, *rest):
    if final:
        cw_ref, cb_ref, out_ref = rest
    else:
        (out_ref,) = rest
    stat = stat_ref[...]
    mu = stat[0:1, :] / jnp.float32(N)
    var = stat[1:2, :] / jnp.float32(N) - mu * mu
    inv = lax.rsqrt(var + 1e-5)
    h = (pre_ref[...] - mu) * (inv * g_ref[...]) + b_ref[...]
    h = jnp.maximum(h, 0.0)
    if final:
        out_ref[...] = (jnp.dot(h, cw_ref[...],
                                preferred_element_type=jnp.float32)
                        + cb_ref[...])
    else:
        out_ref[...] = h


def _make_c2(final):
    grid = N // BR
    rb = lambda i: (i, 0)
    cb = lambda i: (0, 0)
    K = 3 * HID
    in_specs = [
        pl.BlockSpec((BR, K), rb),
        pl.BlockSpec((8, K), cb),
        pl.BlockSpec((1, K), cb),
        pl.BlockSpec((1, K), cb),
    ]
    if final:
        in_specs += [pl.BlockSpec((K, HID), cb), pl.BlockSpec((1, HID), cb)]
        out_w = HID
    else:
        out_w = K
    return pl.pallas_call(
        functools.partial(_c2_body, final),
        grid=(grid,),
        in_specs=in_specs,
        out_specs=pl.BlockSpec((BR, out_w), rb),
        out_shape=jax.ShapeDtypeStruct((N, out_w), jnp.float32),
    )


def _layer_aggregate(tbl2d, ncw, srcl, ldstl, cnt):
    sums, maxs, mins = [], [], []
    for c in range(ncw):
        s, m, n = _make_agg(ncw, c)(tbl2d, srcl, ldstl, cnt)
        sums.append(s.reshape(NPAD, W))
        maxs.append(m.reshape(NPAD, W))
        mins.append(n.reshape(NPAD, W))
    sm = jnp.concatenate(sums, axis=1)[:N]
    mx = jnp.concatenate(maxs, axis=1)[:N]
    mn = jnp.concatenate(mins, axis=1)[:N]
    return sm, mx, mn


def kernel(x, edge_index,
           Wl_0_0, bl_0_0, Wr_0_0,
           Wl_0_1, bl_0_1, Wr_0_1,
           Wl_0_2, bl_0_2, Wr_0_2,
           bn_g_0, bn_b_0,
           Wl_1_0, bl_1_0, Wr_1_0,
           Wl_1_1, bl_1_1, Wr_1_1,
           Wl_1_2, bl_1_2, Wr_1_2,
           bn_g_1, bn_b_1,
           clf_W, clf_b):
    srcl, ldstl, cnt = _bin_edges(edge_index.reshape(2 * E))
    srcl, ldstl, deg = _sort_bins(srcl, ldstl, cnt)

    sm0, mx0, mn0 = _layer_aggregate(x, D_IN // W, srcl, ldstl, cnt)
    degv = deg.reshape(NV, 160)[:, :R].reshape(NPAD, 1)[:N]
    c1 = _make_c1(D_IN)
    pre0, stat0 = c1(x, sm0, mx0, mn0, degv,
                     Wl_0_0, Wr_0_0, Wl_0_1, Wr_0_1, Wl_0_2, Wr_0_2,
                     bl_0_0.reshape(1, HID), bl_0_1.reshape(1, HID),
                     bl_0_2.reshape(1, HID))
    h1 = _make_c2(False)(pre0, stat0, bn_g_0.reshape(1, -1),
                         bn_b_0.reshape(1, -1))

    tbl1 = h1.reshape(N * (3 * HID // W), W)
    sm1, mx1, mn1 = _layer_aggregate(tbl1, 3 * HID // W, srcl, ldstl, cnt)
    c1b = _make_c1(3 * HID)
    pre1, stat1 = c1b(h1, sm1, mx1, mn1, degv,
                      Wl_1_0, Wr_1_0, Wl_1_1, Wr_1_1, Wl_1_2, Wr_1_2,
                      bl_1_0.reshape(1, HID), bl_1_1.reshape(1, HID),
                      bl_1_2.reshape(1, HID))
    clf_Wp = jnp.pad(clf_W, ((0, 0), (0, HID - clf_W.shape[1])))
    clf_bp = jnp.pad(clf_b, (0, HID - clf_b.shape[0])).reshape(1, HID)
    logits = _make_c2(True)(pre1, stat1, bn_g_1.reshape(1, -1),
                            bn_b_1.reshape(1, -1), clf_Wp, clf_bp)
    return logits[:, :clf_W.shape[1]]
